# Initial kernel scaffold; baseline (speedup 1.0000x reference)
import jax
import jax.numpy as jnp
from jax.experimental import pallas as pl


def kernel(x, edge_index, batch, d_row, d_col, d_val, pool_seg, W1, b1, W2, b2, Wf1, bf1, Wf2, bf2):
    raise NotImplementedError("write your pallas kernel here")


# speedup vs baseline: 16.6834x; 16.6834x over previous
import functools

import jax
import jax.numpy as jnp
from jax import lax
from jax.experimental import pallas as pl
from jax.experimental.pallas import tpu as pltpu
from jax.experimental.pallas import tpu_sc as plsc

N = 10000
E = 320000
HID = 128
B = 8
BANDS = 3
M = 960000
R = BANDS * N
NSEG = BANDS * B

NTILES = 16
NCORES = 2

ECH = 2560
E_PAD = ECH * 128
ECH_T = ECH // (NCORES * NTILES)
MCH = 7680
M_PAD = MCH * 128
MCH_T = MCH // (NCORES * NTILES)
MBLK = 48

NTRASH = 16
AROWS = 32

_mesh = plsc.VectorSubcoreMesh(core_axis_name="c", subcore_axis_name="s")


def _zero_vmem_2d(ref, rows, width):
    z = jnp.zeros((16,), jnp.float32)

    def body(i, _):
        for v in range(width // 16):
            ref[i, pl.ds(v * 16, 16)] = z
        return 0

    lax.fori_loop(0, rows, body, 0)


def _zero_vmem_1d(ref, n):
    z = jnp.zeros((16,), jnp.float32)

    def body(i, _):
        ref[pl.ds(i * 16, 16)] = z
        return 0

    lax.fori_loop(0, n // 16, body, 0)


@functools.partial(
    pl.kernel,
    out_type=jax.ShapeDtypeStruct((N,), jnp.float32),
    mesh=_mesh,
    scratch_types=[
        pltpu.VMEM_SHARED((N + NTRASH,), jnp.float32),
        pltpu.VMEM((ECH // NTILES, 128), jnp.int32),
        pltpu.VMEM((640,), jnp.float32),
        pltpu.VMEM((128,), jnp.float32),
    ],
)
def _deg_kernel(row_hbm, deg_hbm, deg_sh, row_v, buf_a, ones_v):
    c = lax.axis_index("c")
    s = lax.axis_index("s")

    @pl.when(c == 0)
    def _():
        _zero_vmem_1d(buf_a, 640)

        @pl.when(s < 15)
        def _():
            pltpu.sync_copy(buf_a, deg_sh.at[pl.ds(s * 640, 640)])

        @pl.when(s == 15)
        def _():
            pltpu.sync_copy(buf_a.at[pl.ds(0, 416)],
                            deg_sh.at[pl.ds(9600, 416)])

        one = jnp.full((16,), 1.0, jnp.float32)
        for v in range(8):
            ones_v[pl.ds(v * 16, 16)] = one
        plsc.subcore_barrier()

        nch = ECH // NTILES
        pltpu.sync_copy(row_hbm.at[pl.ds(s * nch, nch)], row_v)

        def scat(j, _):
            pltpu.sync_copy(ones_v, deg_sh.at[row_v.at[j]], add=True)
            return 0

        lax.fori_loop(0, nch, scat, 0)
        plsc.subcore_barrier()

        @pl.when(s < 15)
        def _():
            pltpu.sync_copy(deg_sh.at[pl.ds(s * 640, 640)], buf_a)
            pltpu.sync_copy(buf_a, deg_hbm.at[pl.ds(s * 640, 640)])

        @pl.when(s == 15)
        def _():
            pltpu.sync_copy(deg_sh.at[pl.ds(9600, 400)],
                            buf_a.at[pl.ds(0, 400)])
            pltpu.sync_copy(buf_a.at[pl.ds(0, 400)],
                            deg_hbm.at[pl.ds(9600, 400)])


@functools.partial(
    pl.kernel,
    out_type=jax.ShapeDtypeStruct((NCORES * AROWS * N,), jnp.float32),
    mesh=_mesh,
    scratch_types=[
        pltpu.VMEM_SHARED((AROWS * N,), jnp.float32),
        pltpu.VMEM((MBLK, 128), jnp.int32),
        pltpu.VMEM((MBLK, 128), jnp.int32),
        pltpu.VMEM((MBLK, 128), jnp.float32),
        pltpu.VMEM((MBLK, 128), jnp.int32),
        pltpu.VMEM((2048,), jnp.float32),
        pltpu.SemaphoreType.DMA,
    ],
)
def _abuild_kernel(drow_hbm, dcol_hbm, dval_hbm, ps2_hbm, out_hbm,
                   a_sh, drow_v, dcol_v, dval_v, seg_v, zbuf, semA):
    c = lax.axis_index("c")
    s = lax.axis_index("s")
    w = c * NTILES + s

    _zero_vmem_1d(zbuf, 2048)
    nwords = AROWS * N // NTILES

    def zb(i, _):
        pltpu.sync_copy(zbuf.at[pl.ds(0, 2000)],
                        a_sh.at[pl.ds(s * nwords + i * 2000, 2000)])
        return 0

    lax.fori_loop(0, nwords // 2000, zb, 0)
    plsc.subcore_barrier()

    for blk in range(MCH_T // MBLK):
        cbase = w * MCH_T + blk * MBLK
        pltpu.sync_copy(drow_hbm.at[pl.ds(cbase, MBLK)], drow_v)
        pltpu.sync_copy(dcol_hbm.at[pl.ds(cbase, MBLK)], dcol_v)
        pltpu.sync_copy(dval_hbm.at[pl.ds(cbase, MBLK)], dval_v)

        def segg(j, _):
            pltpu.async_copy(ps2_hbm.at[drow_v.at[j]], seg_v.at[j], semA)
            return 0

        lax.fori_loop(0, MBLK, segg, 0)

        def segw(j, _):
            pltpu.make_async_copy(ps2_hbm.at[drow_v.at[0]], seg_v.at[0],
                                  semA).wait()
            return 0

        lax.fori_loop(0, MBLK, segw, 0)

        def fidx(j, _):
            for v in range(8):
                seg = seg_v[j, pl.ds(v * 16, 16)]
                dc = dcol_v[j, pl.ds(v * 16, 16)]
                drow_v[j, pl.ds(v * 16, 16)] = dc * AROWS + seg
            return 0

        lax.fori_loop(0, MBLK, fidx, 0)

        def scat(j, _):
            pltpu.sync_copy(dval_v.at[j], a_sh.at[drow_v.at[j]], add=True)
            return 0

        lax.fori_loop(0, MBLK, scat, 0)

    plsc.subcore_barrier()

    def outcp(i, _):
        off = s * nwords + i * 2000
        pltpu.sync_copy(a_sh.at[pl.ds(off, 2000)], zbuf.at[pl.ds(0, 2000)])
        pltpu.sync_copy(zbuf.at[pl.ds(0, 2000)],
                        out_hbm.at[pl.ds(c * (AROWS * N) + off, 2000)])
        return 0

    lax.fori_loop(0, nwords // 2000, outcp, 0)


_AGG_ROWS_T = 640
EC = 64
NCHUNK = E_PAD // EC
NCHUNK_T = NCHUNK // (NCORES * NTILES)


@functools.partial(
    pl.kernel,
    out_type=jax.ShapeDtypeStruct((NCORES, N, HID), jnp.float32),
    mesh=_mesh,
    scratch_types=[
        pltpu.VMEM_SHARED((N + NTRASH, HID), jnp.float32),
        pltpu.VMEM((2, EC), jnp.int32),
        pltpu.VMEM((2, EC), jnp.int32),
        pltpu.VMEM((EC, HID), jnp.float32),
        pltpu.VMEM((EC, HID), jnp.float32),
        pltpu.VMEM((80, HID), jnp.float32),
        pltpu.SemaphoreType.DMA,
        pltpu.SemaphoreType.DMA,
        pltpu.SemaphoreType.DMA,
        pltpu.SemaphoreType.DMA,
    ],
)
def _agg_kernel(ec_hbm, g_hbm, out_hbm, agg_sh, ec0, ec1, gb0, gb1, zbuf,
                semi0, semi1, semg0, semg1):
    c = lax.axis_index("c")
    s = lax.axis_index("s")
    w = c * NTILES + s
    ecb = (ec0, ec1)
    gbufs = (gb0, gb1)
    semi = (semi0, semi1)
    semg = (semg0, semg1)
    cbase = w * NCHUNK_T

    _zero_vmem_2d(zbuf, 80, HID)
    rbase = s * _AGG_ROWS_T

    @pl.when(s < 15)
    def _():
        for i in range(8):
            pltpu.sync_copy(zbuf, agg_sh.at[pl.ds(rbase + i * 80, 80)])

    @pl.when(s == 15)
    def _():
        for i in range(5):
            pltpu.sync_copy(zbuf, agg_sh.at[pl.ds(rbase + i * 80, 80)])
        pltpu.sync_copy(zbuf.at[pl.ds(0, 16)],
                        agg_sh.at[pl.ds(rbase + 400, 16)])

    plsc.subcore_barrier()

    def issue_idx(b, j):
        pltpu.async_copy(ec_hbm.at[cbase + j], ecb[b], semi[b])

    def wait_idx(b):
        pltpu.make_async_copy(ec_hbm.at[0], ecb[b], semi[b]).wait()

    def issue_gather(b, j):
        del j
        pltpu.async_copy(g_hbm.at[ecb[b].at[1]], gbufs[b], semg[b])

    def wait_gather(b):
        pltpu.make_async_copy(g_hbm.at[pl.ds(0, EC)], gbufs[b],
                              semg[b]).wait()

    issue_idx(0, 0)
    issue_idx(1, 1)
    wait_idx(0)
    issue_gather(0, 0)

    def outer(i, _):
        for b in range(2):
            j = i * 2 + b
            nb = 1 - b
            wait_gather(b)
            pltpu.sync_copy(gbufs[b], agg_sh.at[ecb[b].at[0]], add=True)

            @pl.when(j + 2 < NCHUNK_T)
            def _():
                issue_idx(b, j + 2)

            @pl.when(j + 1 < NCHUNK_T)
            def _():
                wait_idx(nb)
                issue_gather(nb, j + 1)
        return 0

    lax.fori_loop(0, NCHUNK_T // 2, outer, 0)
    plsc.subcore_barrier()

    def outcp(nrows):
        for i in range(nrows // 80):
            pltpu.sync_copy(agg_sh.at[pl.ds(rbase + i * 80, 80)], zbuf)
            pltpu.sync_copy(zbuf, out_hbm.at[c, pl.ds(rbase + i * 80, 80)])

    @pl.when(s < 15)
    def _():
        outcp(_AGG_ROWS_T)

    @pl.when(s == 15)
    def _():
        outcp(N - 15 * _AGG_ROWS_T)


_BLK = 1000


def _t1_body(x_ref, w_ref, deg_ref, hx_ref, g_ref, dinv_ref, invdeg_ref):
    hx = jnp.dot(x_ref[...], w_ref[...], preferred_element_type=jnp.float32)
    hx_ref[...] = hx
    deg = deg_ref[...] + 1.0
    dinv = lax.rsqrt(deg)
    dinv_ref[...] = dinv
    invdeg_ref[...] = 1.0 / deg
    g_ref[...] = hx * dinv


def _t1(x, W1, deg2):
    return pl.pallas_call(
        _t1_body,
        grid=(N // _BLK,),
        in_specs=[
            pl.BlockSpec((_BLK, HID), lambda i: (i, 0)),
            pl.BlockSpec((HID, HID), lambda i: (0, 0)),
            pl.BlockSpec((_BLK, 1), lambda i: (i, 0)),
        ],
        out_specs=[
            pl.BlockSpec((_BLK, HID), lambda i: (i, 0)),
            pl.BlockSpec((_BLK, HID), lambda i: (i, 0)),
            pl.BlockSpec((_BLK, 1), lambda i: (i, 0)),
            pl.BlockSpec((_BLK, 1), lambda i: (i, 0)),
        ],
        out_shape=[
            jax.ShapeDtypeStruct((N, HID), jnp.float32),
            jax.ShapeDtypeStruct((N, HID), jnp.float32),
            jax.ShapeDtypeStruct((N, 1), jnp.float32),
            jax.ShapeDtypeStruct((N, 1), jnp.float32),
        ],
    )(x, W1, deg2)


def _t2_body(a0_ref, a1_ref, hx_ref, dinv_ref, invdeg_ref, b_ref, w_ref,
             hx2_ref, g2_ref):
    h1 = dinv_ref[...] * (a0_ref[...] + a1_ref[...]) \
        + hx_ref[...] * invdeg_ref[...] + b_ref[...]
    h1 = jnp.maximum(h1, 0.0)
    hx2 = jnp.dot(h1, w_ref[...], preferred_element_type=jnp.float32)
    hx2_ref[...] = hx2
    g2_ref[...] = hx2 * dinv_ref[...]


def _t2(a0, a1, hx, dinv, invdeg, b1r, W2):
    return pl.pallas_call(
        _t2_body,
        grid=(N // _BLK,),
        in_specs=[
            pl.BlockSpec((_BLK, HID), lambda i: (i, 0)),
            pl.BlockSpec((_BLK, HID), lambda i: (i, 0)),
            pl.BlockSpec((_BLK, HID), lambda i: (i, 0)),
            pl.BlockSpec((_BLK, 1), lambda i: (i, 0)),
            pl.BlockSpec((_BLK, 1), lambda i: (i, 0)),
            pl.BlockSpec((1, HID), lambda i: (0, 0)),
            pl.BlockSpec((HID, HID), lambda i: (0, 0)),
        ],
        out_specs=[
            pl.BlockSpec((_BLK, HID), lambda i: (i, 0)),
            pl.BlockSpec((_BLK, HID), lambda i: (i, 0)),
        ],
        out_shape=[
            jax.ShapeDtypeStruct((N, HID), jnp.float32),
            jax.ShapeDtypeStruct((N, HID), jnp.float32),
        ],
    )(a0, a1, hx, dinv, invdeg, b1r, W2)


def _t3_body(a0_ref, a1_ref, hx_ref, dinv_ref, invdeg_ref, b_ref,
             A0_ref, A1_ref, wf1_ref, bf1_ref, wf2_ref, bf2_ref,
             out_ref, pool_ref):
    k = pl.program_id(0)
    h2 = dinv_ref[...] * (a0_ref[...] + a1_ref[...]) \
        + hx_ref[...] * invdeg_ref[...] + b_ref[...]
    h2 = jnp.maximum(h2, 0.0)
    A = A0_ref[...] + A1_ref[...]
    part = lax.dot_general(A, h2, (((0,), (0,)), ((), ())),
                           preferred_element_type=jnp.float32)

    @pl.when(k == 0)
    def _():
        pool_ref[...] = part

    @pl.when(k > 0)
    def _():
        pool_ref[...] = pool_ref[...] + part

    @pl.when(k == N // _BLK - 1)
    def _():
        p = pool_ref[...]
        acc = bf1_ref[...]
        for band in range(BANDS):
            acc = acc + jnp.dot(p[band * B:(band + 1) * B],
                                wf1_ref[pl.ds(band * HID, HID), :],
                                preferred_element_type=jnp.float32)
        h1h = jnp.maximum(acc, 0.0)
        out_ref[...] = jnp.dot(h1h, wf2_ref[...],
                               preferred_element_type=jnp.float32) \
            + bf2_ref[...]


def _t3(a0, a1, hx2, dinv, invdeg, b2r, A0, A1, Wf1, bf1r, Wf2p, bf2r):
    return pl.pallas_call(
        _t3_body,
        grid=(N // _BLK,),
        in_specs=[
            pl.BlockSpec((_BLK, HID), lambda k: (k, 0)),
            pl.BlockSpec((_BLK, HID), lambda k: (k, 0)),
            pl.BlockSpec((_BLK, HID), lambda k: (k, 0)),
            pl.BlockSpec((_BLK, 1), lambda k: (k, 0)),
            pl.BlockSpec((_BLK, 1), lambda k: (k, 0)),
            pl.BlockSpec((1, HID), lambda k: (0, 0)),
            pl.BlockSpec((_BLK, AROWS), lambda k: (k, 0)),
            pl.BlockSpec((_BLK, AROWS), lambda k: (k, 0)),
            pl.BlockSpec((BANDS * HID, HID), lambda k: (0, 0)),
            pl.BlockSpec((B, HID), lambda k: (0, 0)),
            pl.BlockSpec((HID, HID), lambda k: (0, 0)),
            pl.BlockSpec((B, HID), lambda k: (0, 0)),
        ],
        out_specs=pl.BlockSpec((B, HID), lambda k: (0, 0)),
        out_shape=jax.ShapeDtypeStruct((B, HID), jnp.float32),
        scratch_shapes=[pltpu.VMEM((AROWS, HID), jnp.float32)],
    )(a0, a1, hx2, dinv, invdeg, b2r, A0, A1, Wf1, bf1r, Wf2p, bf2r)


def kernel(x, edge_index, batch, d_row, d_col, d_val, pool_seg,
           W1, b1, W2, b2, Wf1, bf1, Wf2, bf2):
    row, col = edge_index[0], edge_index[1]

    npad_e = E_PAD - E
    pad_i = jnp.arange(npad_e, dtype=jnp.int32)
    row_pad = jnp.concatenate([row, N + (pad_i % NTRASH)])
    col_pad = jnp.concatenate([col, (pad_i * 79) % N])
    row_p = row_pad.reshape(ECH, 128)
    ec = jnp.stack([row_pad.reshape(NCHUNK, EC),
                    col_pad.reshape(NCHUNK, EC)], axis=1)

    npad_m = M_PAD - M
    zpad = jnp.zeros((npad_m,), jnp.int32)
    drow_p = jnp.concatenate([d_row, zpad]).reshape(MCH, 128)
    dcol_p = jnp.concatenate([d_col, zpad]).reshape(MCH, 128)
    dval_p = jnp.concatenate([d_val, zpad.astype(jnp.float32)]).reshape(MCH, 128)

    ps2 = (pool_seg % BANDS) * B + pool_seg // BANDS

    deg = _deg_kernel(row_p)

    Ap = _abuild_kernel(drow_p, dcol_p, dval_p, ps2)
    A0 = Ap[:AROWS * N].reshape(N, AROWS)
    A1 = Ap[AROWS * N:].reshape(N, AROWS)

    hx1, g1, dinv2, invdeg2 = _t1(x, W1, deg.reshape(N, 1))
    agg1 = _agg_kernel(ec, g1)
    hx2, g2 = _t2(agg1[0], agg1[1], hx1, dinv2, invdeg2,
                  b1.reshape(1, HID), W2)
    agg2 = _agg_kernel(ec, g2)

    Wf2p = jnp.pad(Wf2, ((0, 0), (0, HID - Wf2.shape[1])))
    bf2r = jnp.pad(bf2, (0, HID - bf2.shape[0])).reshape(1, HID)
    bf2b = jnp.broadcast_to(bf2r, (B, HID))
    bf1r = jnp.broadcast_to(bf1.reshape(1, HID), (B, HID))

    outp = _t3(agg2[0], agg2[1], hx2, dinv2, invdeg2, b2.reshape(1, HID),
               A0, A1, Wf1, bf1r, Wf2p, bf2b)
    return outp[:, :Wf2.shape[1]]


# speedup vs baseline: 20.7531x; 1.2439x over previous
import functools

import jax
import jax.numpy as jnp
from jax import lax
from jax.experimental import pallas as pl
from jax.experimental.pallas import tpu as pltpu
from jax.experimental.pallas import tpu_sc as plsc

N = 10000
E = 320000
HID = 128
B = 8
BANDS = 3
M = 960000
R = BANDS * N
NSEG = BANDS * B

NTILES = 16
NCORES = 2

ECH = 2560
E_PAD = ECH * 128
ECH_T = ECH // (NCORES * NTILES)
MCH = 7680
M_PAD = MCH * 128
MCH_T = MCH // (NCORES * NTILES)
MBLK = 48

NTRASH = 16
AROWS = 32

_mesh = plsc.VectorSubcoreMesh(core_axis_name="c", subcore_axis_name="s")


def _zero_vmem_2d(ref, rows, width):
    z = jnp.zeros((16,), jnp.float32)

    def body(i, _):
        for v in range(width // 16):
            ref[i, pl.ds(v * 16, 16)] = z
        return 0

    lax.fori_loop(0, rows, body, 0)


def _zero_vmem_1d(ref, n):
    z = jnp.zeros((16,), jnp.float32)

    def body(i, _):
        ref[pl.ds(i * 16, 16)] = z
        return 0

    lax.fori_loop(0, n // 16, body, 0)


@functools.partial(
    pl.kernel,
    out_type=jax.ShapeDtypeStruct((N,), jnp.float32),
    mesh=_mesh,
    scratch_types=[
        pltpu.VMEM_SHARED((N + NTRASH,), jnp.float32),
        pltpu.VMEM((ECH // NTILES, 128), jnp.int32),
        pltpu.VMEM((640,), jnp.float32),
        pltpu.VMEM((128,), jnp.float32),
    ],
)
def _deg_kernel(row_hbm, deg_hbm, deg_sh, row_v, buf_a, ones_v):
    c = lax.axis_index("c")
    s = lax.axis_index("s")

    @pl.when(c == 0)
    def _():
        _zero_vmem_1d(buf_a, 640)

        @pl.when(s < 15)
        def _():
            pltpu.sync_copy(buf_a, deg_sh.at[pl.ds(s * 640, 640)])

        @pl.when(s == 15)
        def _():
            pltpu.sync_copy(buf_a.at[pl.ds(0, 416)],
                            deg_sh.at[pl.ds(9600, 416)])

        one = jnp.full((16,), 1.0, jnp.float32)
        for v in range(8):
            ones_v[pl.ds(v * 16, 16)] = one
        plsc.subcore_barrier()

        nch = ECH // NTILES
        pltpu.sync_copy(row_hbm.at[pl.ds(s * nch, nch)], row_v)

        def scat(j, _):
            pltpu.sync_copy(ones_v, deg_sh.at[row_v.at[j]], add=True)
            return 0

        lax.fori_loop(0, nch, scat, 0)
        plsc.subcore_barrier()

        @pl.when(s < 15)
        def _():
            pltpu.sync_copy(deg_sh.at[pl.ds(s * 640, 640)], buf_a)
            pltpu.sync_copy(buf_a, deg_hbm.at[pl.ds(s * 640, 640)])

        @pl.when(s == 15)
        def _():
            pltpu.sync_copy(deg_sh.at[pl.ds(9600, 400)],
                            buf_a.at[pl.ds(0, 400)])
            pltpu.sync_copy(buf_a.at[pl.ds(0, 400)],
                            deg_hbm.at[pl.ds(9600, 400)])


_NBLK = MCH_T // MBLK


@functools.partial(
    pl.kernel,
    out_type=jax.ShapeDtypeStruct((NCORES * AROWS * N,), jnp.float32),
    mesh=_mesh,
    scratch_types=[
        pltpu.VMEM_SHARED((AROWS * N,), jnp.float32),
        pltpu.VMEM_SHARED((R,), jnp.int32),
        pltpu.VMEM((MBLK, 128), jnp.int32),
        pltpu.VMEM((MBLK, 128), jnp.int32),
        pltpu.VMEM((MBLK, 128), jnp.int32),
        pltpu.VMEM((MBLK, 128), jnp.int32),
        pltpu.VMEM((MBLK, 128), jnp.float32),
        pltpu.VMEM((MBLK, 128), jnp.float32),
        pltpu.VMEM((MBLK, 128), jnp.int32),
        pltpu.VMEM((1880,), jnp.int32),
        pltpu.VMEM((2048,), jnp.float32),
        pltpu.SemaphoreType.DMA,
        pltpu.SemaphoreType.DMA,
        pltpu.SemaphoreType.DMA,
    ],
)
def _abuild_kernel(drow_hbm, dcol_hbm, dval_hbm, ps2_hbm, out_hbm,
                   a_sh, ps2_sh, dr0, dr1, dc0, dc1, dv0, dv1, seg_v, tbuf,
                   zbuf, semL0, semL1, semS):
    c = lax.axis_index("c")
    s = lax.axis_index("s")
    w = c * NTILES + s
    drb = (dr0, dr1)
    dcb = (dc0, dc1)
    dvb = (dv0, dv1)
    semL = (semL0, semL1)

    @pl.when(s < 15)
    def _():
        pltpu.sync_copy(ps2_hbm.at[pl.ds(s * 1880, 1880)], tbuf)
        pltpu.sync_copy(tbuf, ps2_sh.at[pl.ds(s * 1880, 1880)])

    @pl.when(s == 15)
    def _():
        pltpu.sync_copy(ps2_hbm.at[pl.ds(28200, 1800)],
                        tbuf.at[pl.ds(0, 1800)])
        pltpu.sync_copy(tbuf.at[pl.ds(0, 1800)],
                        ps2_sh.at[pl.ds(28200, 1800)])

    _zero_vmem_1d(zbuf, 2048)
    nwords = AROWS * N // NTILES

    def zb(i, _):
        pltpu.sync_copy(zbuf.at[pl.ds(0, 2000)],
                        a_sh.at[pl.ds(s * nwords + i * 2000, 2000)])
        return 0

    lax.fori_loop(0, nwords // 2000, zb, 0)
    plsc.subcore_barrier()

    def issue_loads(b, blk):
        cbase = w * MCH_T + blk * MBLK
        pltpu.async_copy(drow_hbm.at[pl.ds(cbase, MBLK)], drb[b], semL[b])
        pltpu.async_copy(dcol_hbm.at[pl.ds(cbase, MBLK)], dcb[b], semL[b])
        pltpu.async_copy(dval_hbm.at[pl.ds(cbase, MBLK)], dvb[b], semL[b])

    def wait_loads(b):
        pltpu.make_async_copy(drow_hbm.at[pl.ds(0, MBLK)], drb[b],
                              semL[b]).wait()
        pltpu.make_async_copy(dcol_hbm.at[pl.ds(0, MBLK)], dcb[b],
                              semL[b]).wait()
        pltpu.make_async_copy(dval_hbm.at[pl.ds(0, MBLK)], dvb[b],
                              semL[b]).wait()

    issue_loads(0, 0)
    for blk in range(_NBLK):
        b = blk % 2
        wait_loads(b)
        if blk + 1 < _NBLK:
            issue_loads(1 - b, blk + 1)

        def chunk(j, _):
            pltpu.sync_copy(ps2_sh.at[drb[b].at[j]], seg_v.at[j])
            for v in range(8):
                seg = seg_v[j, pl.ds(v * 16, 16)]
                dc = dcb[b][j, pl.ds(v * 16, 16)]
                drb[b][j, pl.ds(v * 16, 16)] = dc * AROWS + seg
            pltpu.async_copy(dvb[b].at[j], a_sh.at[drb[b].at[j]], semS,
                             add=True)
            return 0

        lax.fori_loop(0, MBLK, chunk, 0)

        def drain(j, _):
            pltpu.make_async_copy(dvb[b].at[0], a_sh.at[drb[b].at[0]],
                                  semS).wait()
            return 0

        lax.fori_loop(0, MBLK, drain, 0)

    plsc.subcore_barrier()

    def outcp(i, _):
        off = s * nwords + i * 2000
        pltpu.sync_copy(a_sh.at[pl.ds(off, 2000)], zbuf.at[pl.ds(0, 2000)])
        pltpu.sync_copy(zbuf.at[pl.ds(0, 2000)],
                        out_hbm.at[pl.ds(c * (AROWS * N) + off, 2000)])
        return 0

    lax.fori_loop(0, nwords // 2000, outcp, 0)


_AGG_ROWS_T = 640
EC = 64
NCHUNK = E_PAD // EC
NCHUNK_T = NCHUNK // (NCORES * NTILES)


@functools.partial(
    pl.kernel,
    out_type=jax.ShapeDtypeStruct((NCORES, N, HID), jnp.float32),
    mesh=_mesh,
    scratch_types=[
        pltpu.VMEM_SHARED((N + NTRASH, HID), jnp.float32),
        pltpu.VMEM((2, EC), jnp.int32),
        pltpu.VMEM((2, EC), jnp.int32),
        pltpu.VMEM((EC, HID), jnp.float32),
        pltpu.VMEM((EC, HID), jnp.float32),
        pltpu.VMEM((80, HID), jnp.float32),
        pltpu.SemaphoreType.DMA,
        pltpu.SemaphoreType.DMA,
        pltpu.SemaphoreType.DMA,
        pltpu.SemaphoreType.DMA,
    ],
)
def _agg_kernel(ec_hbm, g_hbm, out_hbm, agg_sh, ec0, ec1, gb0, gb1, zbuf,
                semi0, semi1, semg0, semg1):
    c = lax.axis_index("c")
    s = lax.axis_index("s")
    w = c * NTILES + s
    ecb = (ec0, ec1)
    gbufs = (gb0, gb1)
    semi = (semi0, semi1)
    semg = (semg0, semg1)
    cbase = w * NCHUNK_T

    _zero_vmem_2d(zbuf, 80, HID)
    rbase = s * _AGG_ROWS_T

    @pl.when(s < 15)
    def _():
        for i in range(8):
            pltpu.sync_copy(zbuf, agg_sh.at[pl.ds(rbase + i * 80, 80)])

    @pl.when(s == 15)
    def _():
        for i in range(5):
            pltpu.sync_copy(zbuf, agg_sh.at[pl.ds(rbase + i * 80, 80)])
        pltpu.sync_copy(zbuf.at[pl.ds(0, 16)],
                        agg_sh.at[pl.ds(rbase + 400, 16)])

    plsc.subcore_barrier()

    def issue_idx(b, j):
        pltpu.async_copy(ec_hbm.at[cbase + j], ecb[b], semi[b])

    def wait_idx(b):
        pltpu.make_async_copy(ec_hbm.at[0], ecb[b], semi[b]).wait()

    def issue_gather(b, j):
        del j
        pltpu.async_copy(g_hbm.at[ecb[b].at[1]], gbufs[b], semg[b])

    def wait_gather(b):
        pltpu.make_async_copy(g_hbm.at[pl.ds(0, EC)], gbufs[b],
                              semg[b]).wait()

    issue_idx(0, 0)
    issue_idx(1, 1)
    wait_idx(0)
    issue_gather(0, 0)

    def outer(i, _):
        for b in range(2):
            j = i * 2 + b
            nb = 1 - b
            wait_gather(b)
            pltpu.sync_copy(gbufs[b], agg_sh.at[ecb[b].at[0]], add=True)

            @pl.when(j + 2 < NCHUNK_T)
            def _():
                issue_idx(b, j + 2)

            @pl.when(j + 1 < NCHUNK_T)
            def _():
                wait_idx(nb)
                issue_gather(nb, j + 1)
        return 0

    lax.fori_loop(0, NCHUNK_T // 2, outer, 0)
    plsc.subcore_barrier()

    def outcp(nrows):
        for i in range(nrows // 80):
            pltpu.sync_copy(agg_sh.at[pl.ds(rbase + i * 80, 80)], zbuf)
            pltpu.sync_copy(zbuf, out_hbm.at[c, pl.ds(rbase + i * 80, 80)])

    @pl.when(s < 15)
    def _():
        outcp(_AGG_ROWS_T)

    @pl.when(s == 15)
    def _():
        outcp(N - 15 * _AGG_ROWS_T)


_BLK = 1000


def _t1_body(x_ref, w_ref, deg_ref, hx_ref, g_ref, dinv_ref, invdeg_ref):
    hx = jnp.dot(x_ref[...], w_ref[...], preferred_element_type=jnp.float32)
    hx_ref[...] = hx
    deg = deg_ref[...] + 1.0
    dinv = lax.rsqrt(deg)
    dinv_ref[...] = dinv
    invdeg_ref[...] = 1.0 / deg
    g_ref[...] = hx * dinv


def _t1(x, W1, deg2):
    return pl.pallas_call(
        _t1_body,
        grid=(N // _BLK,),
        in_specs=[
            pl.BlockSpec((_BLK, HID), lambda i: (i, 0)),
            pl.BlockSpec((HID, HID), lambda i: (0, 0)),
            pl.BlockSpec((_BLK, 1), lambda i: (i, 0)),
        ],
        out_specs=[
            pl.BlockSpec((_BLK, HID), lambda i: (i, 0)),
            pl.BlockSpec((_BLK, HID), lambda i: (i, 0)),
            pl.BlockSpec((_BLK, 1), lambda i: (i, 0)),
            pl.BlockSpec((_BLK, 1), lambda i: (i, 0)),
        ],
        out_shape=[
            jax.ShapeDtypeStruct((N, HID), jnp.float32),
            jax.ShapeDtypeStruct((N, HID), jnp.float32),
            jax.ShapeDtypeStruct((N, 1), jnp.float32),
            jax.ShapeDtypeStruct((N, 1), jnp.float32),
        ],
    )(x, W1, deg2)


def _t2_body(a0_ref, a1_ref, hx_ref, dinv_ref, invdeg_ref, b_ref, w_ref,
             hx2_ref, g2_ref):
    h1 = dinv_ref[...] * (a0_ref[...] + a1_ref[...]) \
        + hx_ref[...] * invdeg_ref[...] + b_ref[...]
    h1 = jnp.maximum(h1, 0.0)
    hx2 = jnp.dot(h1, w_ref[...], preferred_element_type=jnp.float32)
    hx2_ref[...] = hx2
    g2_ref[...] = hx2 * dinv_ref[...]


def _t2(a0, a1, hx, dinv, invdeg, b1r, W2):
    return pl.pallas_call(
        _t2_body,
        grid=(N // _BLK,),
        in_specs=[
            pl.BlockSpec((_BLK, HID), lambda i: (i, 0)),
            pl.BlockSpec((_BLK, HID), lambda i: (i, 0)),
            pl.BlockSpec((_BLK, HID), lambda i: (i, 0)),
            pl.BlockSpec((_BLK, 1), lambda i: (i, 0)),
            pl.BlockSpec((_BLK, 1), lambda i: (i, 0)),
            pl.BlockSpec((1, HID), lambda i: (0, 0)),
            pl.BlockSpec((HID, HID), lambda i: (0, 0)),
        ],
        out_specs=[
            pl.BlockSpec((_BLK, HID), lambda i: (i, 0)),
            pl.BlockSpec((_BLK, HID), lambda i: (i, 0)),
        ],
        out_shape=[
            jax.ShapeDtypeStruct((N, HID), jnp.float32),
            jax.ShapeDtypeStruct((N, HID), jnp.float32),
        ],
    )(a0, a1, hx, dinv, invdeg, b1r, W2)


def _t3_body(a0_ref, a1_ref, hx_ref, dinv_ref, invdeg_ref, b_ref,
             A0_ref, A1_ref, wf1_ref, bf1_ref, wf2_ref, bf2_ref,
             out_ref, pool_ref):
    k = pl.program_id(0)
    h2 = dinv_ref[...] * (a0_ref[...] + a1_ref[...]) \
        + hx_ref[...] * invdeg_ref[...] + b_ref[...]
    h2 = jnp.maximum(h2, 0.0)
    A = A0_ref[...] + A1_ref[...]
    part = lax.dot_general(A, h2, (((0,), (0,)), ((), ())),
                           preferred_element_type=jnp.float32)

    @pl.when(k == 0)
    def _():
        pool_ref[...] = part

    @pl.when(k > 0)
    def _():
        pool_ref[...] = pool_ref[...] + part

    @pl.when(k == N // _BLK - 1)
    def _():
        p = pool_ref[...]
        acc = bf1_ref[...]
        for band in range(BANDS):
            acc = acc + jnp.dot(p[band * B:(band + 1) * B],
                                wf1_ref[pl.ds(band * HID, HID), :],
                                preferred_element_type=jnp.float32)
        h1h = jnp.maximum(acc, 0.0)
        out_ref[...] = jnp.dot(h1h, wf2_ref[...],
                               preferred_element_type=jnp.float32) \
            + bf2_ref[...]


def _t3(a0, a1, hx2, dinv, invdeg, b2r, A0, A1, Wf1, bf1r, Wf2p, bf2r):
    return pl.pallas_call(
        _t3_body,
        grid=(N // _BLK,),
        in_specs=[
            pl.BlockSpec((_BLK, HID), lambda k: (k, 0)),
            pl.BlockSpec((_BLK, HID), lambda k: (k, 0)),
            pl.BlockSpec((_BLK, HID), lambda k: (k, 0)),
            pl.BlockSpec((_BLK, 1), lambda k: (k, 0)),
            pl.BlockSpec((_BLK, 1), lambda k: (k, 0)),
            pl.BlockSpec((1, HID), lambda k: (0, 0)),
            pl.BlockSpec((_BLK, AROWS), lambda k: (k, 0)),
            pl.BlockSpec((_BLK, AROWS), lambda k: (k, 0)),
            pl.BlockSpec((BANDS * HID, HID), lambda k: (0, 0)),
            pl.BlockSpec((B, HID), lambda k: (0, 0)),
            pl.BlockSpec((HID, HID), lambda k: (0, 0)),
            pl.BlockSpec((B, HID), lambda k: (0, 0)),
        ],
        out_specs=pl.BlockSpec((B, HID), lambda k: (0, 0)),
        out_shape=jax.ShapeDtypeStruct((B, HID), jnp.float32),
        scratch_shapes=[pltpu.VMEM((AROWS, HID), jnp.float32)],
    )(a0, a1, hx2, dinv, invdeg, b2r, A0, A1, Wf1, bf1r, Wf2p, bf2r)


def kernel(x, edge_index, batch, d_row, d_col, d_val, pool_seg,
           W1, b1, W2, b2, Wf1, bf1, Wf2, bf2):
    row, col = edge_index[0], edge_index[1]

    npad_e = E_PAD - E
    pad_i = jnp.arange(npad_e, dtype=jnp.int32)
    row_pad = jnp.concatenate([row, N + (pad_i % NTRASH)])
    col_pad = jnp.concatenate([col, (pad_i * 79) % N])
    row_p = row_pad.reshape(ECH, 128)
    ec = jnp.stack([row_pad.reshape(NCHUNK, EC),
                    col_pad.reshape(NCHUNK, EC)], axis=1)

    npad_m = M_PAD - M
    zpad = jnp.zeros((npad_m,), jnp.int32)
    drow_p = jnp.concatenate([d_row, zpad]).reshape(MCH, 128)
    dcol_p = jnp.concatenate([d_col, zpad]).reshape(MCH, 128)
    dval_p = jnp.concatenate([d_val, zpad.astype(jnp.float32)]).reshape(MCH, 128)

    ps2 = (pool_seg % BANDS) * B + pool_seg // BANDS

    deg = _deg_kernel(row_p)

    Ap = _abuild_kernel(drow_p, dcol_p, dval_p, ps2)
    A0 = Ap[:AROWS * N].reshape(N, AROWS)
    A1 = Ap[AROWS * N:].reshape(N, AROWS)

    hx1, g1, dinv2, invdeg2 = _t1(x, W1, deg.reshape(N, 1))
    agg1 = _agg_kernel(ec, g1)
    hx2, g2 = _t2(agg1[0], agg1[1], hx1, dinv2, invdeg2,
                  b1.reshape(1, HID), W2)
    agg2 = _agg_kernel(ec, g2)

    Wf2p = jnp.pad(Wf2, ((0, 0), (0, HID - Wf2.shape[1])))
    bf2r = jnp.pad(bf2, (0, HID - bf2.shape[0])).reshape(1, HID)
    bf2b = jnp.broadcast_to(bf2r, (B, HID))
    bf1r = jnp.broadcast_to(bf1.reshape(1, HID), (B, HID))

    outp = _t3(agg2[0], agg2[1], hx2, dinv2, invdeg2, b2.reshape(1, HID),
               A0, A1, Wf1, bf1r, Wf2p, bf2b)
    return outp[:, :Wf2.shape[1]]


# speedup vs baseline: 24.9376x; 1.2016x over previous
import functools

import jax
import jax.numpy as jnp
from jax import lax
from jax.experimental import pallas as pl
from jax.experimental.pallas import tpu as pltpu
from jax.experimental.pallas import tpu_sc as plsc

N = 10000
E = 320000
HID = 128
B = 8
BANDS = 3
M = 960000
R = BANDS * N
NSEG = BANDS * B

NTILES = 16
NCORES = 2

ECH = 2560
E_PAD = ECH * 128
ECH_T = ECH // (NCORES * NTILES)
MCH = 7680
M_PAD = MCH * 128
MCH_T = MCH // (NCORES * NTILES)
MBLK = 48

NTRASH = 16
AROWS = 32

_mesh = plsc.VectorSubcoreMesh(core_axis_name="c", subcore_axis_name="s")


def _zero_vmem_2d(ref, rows, width):
    z = jnp.zeros((16,), jnp.float32)

    def body(i, _):
        for v in range(width // 16):
            ref[i, pl.ds(v * 16, 16)] = z
        return 0

    lax.fori_loop(0, rows, body, 0)


def _zero_vmem_1d(ref, n):
    z = jnp.zeros((16,), jnp.float32)

    def body(i, _):
        ref[pl.ds(i * 16, 16)] = z
        return 0

    lax.fori_loop(0, n // 16, body, 0)


@functools.partial(
    pl.kernel,
    out_type=jax.ShapeDtypeStruct((N,), jnp.float32),
    mesh=_mesh,
    scratch_types=[
        pltpu.VMEM_SHARED((N + NTRASH,), jnp.float32),
        pltpu.VMEM((ECH // NTILES, 128), jnp.int32),
        pltpu.VMEM((640,), jnp.float32),
        pltpu.VMEM((128,), jnp.float32),
    ],
)
def _deg_kernel(row_hbm, deg_hbm, deg_sh, row_v, buf_a, ones_v):
    c = lax.axis_index("c")
    s = lax.axis_index("s")

    @pl.when(c == 0)
    def _():
        _zero_vmem_1d(buf_a, 640)

        @pl.when(s < 15)
        def _():
            pltpu.sync_copy(buf_a, deg_sh.at[pl.ds(s * 640, 640)])

        @pl.when(s == 15)
        def _():
            pltpu.sync_copy(buf_a.at[pl.ds(0, 416)],
                            deg_sh.at[pl.ds(9600, 416)])

        one = jnp.full((16,), 1.0, jnp.float32)
        for v in range(8):
            ones_v[pl.ds(v * 16, 16)] = one
        plsc.subcore_barrier()

        nch = ECH // NTILES
        pltpu.sync_copy(row_hbm.at[pl.ds(s * nch, nch)], row_v)

        def scat(j, _):
            pltpu.sync_copy(ones_v, deg_sh.at[row_v.at[j]], add=True)
            return 0

        lax.fori_loop(0, nch, scat, 0)
        plsc.subcore_barrier()

        @pl.when(s < 15)
        def _():
            pltpu.sync_copy(deg_sh.at[pl.ds(s * 640, 640)], buf_a)
            pltpu.sync_copy(buf_a, deg_hbm.at[pl.ds(s * 640, 640)])

        @pl.when(s == 15)
        def _():
            pltpu.sync_copy(deg_sh.at[pl.ds(9600, 400)],
                            buf_a.at[pl.ds(0, 400)])
            pltpu.sync_copy(buf_a.at[pl.ds(0, 400)],
                            deg_hbm.at[pl.ds(9600, 400)])


_NBLK = MCH_T // MBLK


@functools.partial(
    pl.kernel,
    out_type=jax.ShapeDtypeStruct((NCORES * AROWS * N,), jnp.float32),
    mesh=_mesh,
    scratch_types=[
        pltpu.VMEM_SHARED((AROWS * N,), jnp.float32),
        pltpu.VMEM_SHARED((R,), jnp.int32),
        pltpu.VMEM((MBLK, 128), jnp.int32),
        pltpu.VMEM((MBLK, 128), jnp.int32),
        pltpu.VMEM((MBLK, 128), jnp.int32),
        pltpu.VMEM((MBLK, 128), jnp.int32),
        pltpu.VMEM((MBLK, 128), jnp.float32),
        pltpu.VMEM((MBLK, 128), jnp.float32),
        pltpu.VMEM((MBLK, 128), jnp.int32),
        pltpu.VMEM((1880,), jnp.int32),
        pltpu.VMEM((2048,), jnp.float32),
        pltpu.SemaphoreType.DMA,
        pltpu.SemaphoreType.DMA,
        pltpu.SemaphoreType.DMA,
    ],
)
def _abuild_kernel(drow_hbm, dcol_hbm, dval_hbm, ps2_hbm, out_hbm,
                   a_sh, ps2_sh, dr0, dr1, dc0, dc1, dv0, dv1, seg_v, tbuf,
                   zbuf, semL0, semL1, semS):
    c = lax.axis_index("c")
    s = lax.axis_index("s")
    w = c * NTILES + s
    drb = (dr0, dr1)
    dcb = (dc0, dc1)
    dvb = (dv0, dv1)
    semL = (semL0, semL1)

    @pl.when(s < 15)
    def _():
        pltpu.sync_copy(ps2_hbm.at[pl.ds(s * 1880, 1880)], tbuf)
        pltpu.sync_copy(tbuf, ps2_sh.at[pl.ds(s * 1880, 1880)])

    @pl.when(s == 15)
    def _():
        pltpu.sync_copy(ps2_hbm.at[pl.ds(28200, 1800)],
                        tbuf.at[pl.ds(0, 1800)])
        pltpu.sync_copy(tbuf.at[pl.ds(0, 1800)],
                        ps2_sh.at[pl.ds(28200, 1800)])

    _zero_vmem_1d(zbuf, 2048)
    nwords = AROWS * N // NTILES

    def zb(i, _):
        pltpu.sync_copy(zbuf.at[pl.ds(0, 2000)],
                        a_sh.at[pl.ds(s * nwords + i * 2000, 2000)])
        return 0

    lax.fori_loop(0, nwords // 2000, zb, 0)
    plsc.subcore_barrier()

    def issue_loads(b, blk):
        cbase = w * MCH_T + blk * MBLK
        pltpu.async_copy(drow_hbm.at[pl.ds(cbase, MBLK)], drb[b], semL[b])
        pltpu.async_copy(dcol_hbm.at[pl.ds(cbase, MBLK)], dcb[b], semL[b])
        pltpu.async_copy(dval_hbm.at[pl.ds(cbase, MBLK)], dvb[b], semL[b])

    def wait_loads(b):
        pltpu.make_async_copy(drow_hbm.at[pl.ds(0, MBLK)], drb[b],
                              semL[b]).wait()
        pltpu.make_async_copy(dcol_hbm.at[pl.ds(0, MBLK)], dcb[b],
                              semL[b]).wait()
        pltpu.make_async_copy(dval_hbm.at[pl.ds(0, MBLK)], dvb[b],
                              semL[b]).wait()

    issue_loads(0, 0)
    for blk in range(_NBLK):
        b = blk % 2
        wait_loads(b)
        if blk + 1 < _NBLK:
            issue_loads(1 - b, blk + 1)

        def chunk(j, _):
            pltpu.sync_copy(ps2_sh.at[drb[b].at[j]], seg_v.at[j])
            for v in range(8):
                seg = seg_v[j, pl.ds(v * 16, 16)]
                dc = dcb[b][j, pl.ds(v * 16, 16)]
                drb[b][j, pl.ds(v * 16, 16)] = dc * AROWS + seg
            pltpu.async_copy(dvb[b].at[j], a_sh.at[drb[b].at[j]], semS,
                             add=True)
            return 0

        lax.fori_loop(0, MBLK, chunk, 0)

        def drain(j, _):
            pltpu.make_async_copy(dvb[b].at[0], a_sh.at[drb[b].at[0]],
                                  semS).wait()
            return 0

        lax.fori_loop(0, MBLK, drain, 0)

    plsc.subcore_barrier()

    def outcp(i, _):
        off = s * nwords + i * 2000
        pltpu.sync_copy(a_sh.at[pl.ds(off, 2000)], zbuf.at[pl.ds(0, 2000)])
        pltpu.sync_copy(zbuf.at[pl.ds(0, 2000)],
                        out_hbm.at[pl.ds(c * (AROWS * N) + off, 2000)])
        return 0

    lax.fori_loop(0, nwords // 2000, outcp, 0)


_AGG_ROWS_T = 640
EC = 64
NCHUNK = E_PAD // EC
NCHUNK_T = NCHUNK // (NCORES * NTILES)


@functools.partial(
    pl.kernel,
    out_type=jax.ShapeDtypeStruct((NCORES, N, HID), jnp.float32),
    mesh=_mesh,
    scratch_types=[
        pltpu.VMEM_SHARED((N + NTRASH, HID), jnp.float32),
        pltpu.VMEM((2, EC), jnp.int32),
        pltpu.VMEM((2, EC), jnp.int32),
        pltpu.VMEM((2, EC), jnp.int32),
        pltpu.VMEM((EC, HID), jnp.float32),
        pltpu.VMEM((EC, HID), jnp.float32),
        pltpu.VMEM((80, HID), jnp.float32),
        pltpu.SemaphoreType.DMA,
        pltpu.SemaphoreType.DMA,
        pltpu.SemaphoreType.DMA,
        pltpu.SemaphoreType.DMA,
        pltpu.SemaphoreType.DMA,
    ],
)
def _agg_kernel(ec_hbm, g_hbm, out_hbm, agg_sh, ec0, ec1, sidx, gb0, gb1,
                zbuf, semi0, semi1, semg0, semg1, semS):
    c = lax.axis_index("c")
    s = lax.axis_index("s")
    w = c * NTILES + s
    ecb = (ec0, ec1)
    gbufs = (gb0, gb1)
    semi = (semi0, semi1)
    semg = (semg0, semg1)
    cbase = w * NCHUNK_T

    _zero_vmem_2d(zbuf, 80, HID)
    rbase = s * _AGG_ROWS_T

    @pl.when(s < 15)
    def _():
        for i in range(8):
            pltpu.sync_copy(zbuf, agg_sh.at[pl.ds(rbase + i * 80, 80)])

    @pl.when(s == 15)
    def _():
        for i in range(5):
            pltpu.sync_copy(zbuf, agg_sh.at[pl.ds(rbase + i * 80, 80)])
        pltpu.sync_copy(zbuf.at[pl.ds(0, 16)],
                        agg_sh.at[pl.ds(rbase + 400, 16)])

    plsc.subcore_barrier()

    def issue_idx(b, j):
        pltpu.async_copy(ec_hbm.at[cbase + j], ecb[b], semi[b])

    def wait_idx(b):
        pltpu.make_async_copy(ec_hbm.at[0], ecb[b], semi[b]).wait()

    def issue_gather(b, j):
        del j
        pltpu.async_copy(g_hbm.at[ecb[b].at[1]], gbufs[b], semg[b])

    def wait_gather(b):
        pltpu.make_async_copy(g_hbm.at[pl.ds(0, EC)], gbufs[b],
                              semg[b]).wait()

    def issue_scat(b):
        pltpu.async_copy(gbufs[b], agg_sh.at[sidx.at[b]], semS, add=True)

    def drain_scat(b):
        pltpu.make_async_copy(gbufs[b], agg_sh.at[sidx.at[b]], semS).wait()

    issue_idx(0, 0)
    issue_idx(1, 1)
    wait_idx(0)
    issue_gather(0, 0)

    def outer(i, _):
        for b in range(2):
            j = i * 2 + b
            nb = 1 - b
            wait_gather(b)
            for v in range(EC // 16):
                sidx[b, pl.ds(v * 16, 16)] = ecb[b][0, pl.ds(v * 16, 16)]

            @pl.when(j + 2 < NCHUNK_T)
            def _():
                issue_idx(b, j + 2)

            @pl.when(j >= 1)
            def _():
                drain_scat(nb)

            issue_scat(b)

            @pl.when(j + 1 < NCHUNK_T)
            def _():
                wait_idx(nb)
                issue_gather(nb, j + 1)
        return 0

    lax.fori_loop(0, NCHUNK_T // 2, outer, 0)
    drain_scat(1)
    plsc.subcore_barrier()

    def outcp(nrows):
        for i in range(nrows // 80):
            pltpu.sync_copy(agg_sh.at[pl.ds(rbase + i * 80, 80)], zbuf)
            pltpu.sync_copy(zbuf, out_hbm.at[c, pl.ds(rbase + i * 80, 80)])

    @pl.when(s < 15)
    def _():
        outcp(_AGG_ROWS_T)

    @pl.when(s == 15)
    def _():
        outcp(N - 15 * _AGG_ROWS_T)


_BLK = 1000


def _t1_body(x_ref, w_ref, deg_ref, hx_ref, g_ref, dinv_ref, invdeg_ref):
    hx = jnp.dot(x_ref[...], w_ref[...], preferred_element_type=jnp.float32)
    hx_ref[...] = hx
    deg = deg_ref[...] + 1.0
    dinv = lax.rsqrt(deg)
    dinv_ref[...] = dinv
    invdeg_ref[...] = 1.0 / deg
    g_ref[...] = hx * dinv


def _t1(x, W1, deg2):
    return pl.pallas_call(
        _t1_body,
        grid=(N // _BLK,),
        in_specs=[
            pl.BlockSpec((_BLK, HID), lambda i: (i, 0)),
            pl.BlockSpec((HID, HID), lambda i: (0, 0)),
            pl.BlockSpec((_BLK, 1), lambda i: (i, 0)),
        ],
        out_specs=[
            pl.BlockSpec((_BLK, HID), lambda i: (i, 0)),
            pl.BlockSpec((_BLK, HID), lambda i: (i, 0)),
            pl.BlockSpec((_BLK, 1), lambda i: (i, 0)),
            pl.BlockSpec((_BLK, 1), lambda i: (i, 0)),
        ],
        out_shape=[
            jax.ShapeDtypeStruct((N, HID), jnp.float32),
            jax.ShapeDtypeStruct((N, HID), jnp.float32),
            jax.ShapeDtypeStruct((N, 1), jnp.float32),
            jax.ShapeDtypeStruct((N, 1), jnp.float32),
        ],
    )(x, W1, deg2)


def _t2_body(a0_ref, a1_ref, hx_ref, dinv_ref, invdeg_ref, b_ref, w_ref,
             hx2_ref, g2_ref):
    h1 = dinv_ref[...] * (a0_ref[...] + a1_ref[...]) \
        + hx_ref[...] * invdeg_ref[...] + b_ref[...]
    h1 = jnp.maximum(h1, 0.0)
    hx2 = jnp.dot(h1, w_ref[...], preferred_element_type=jnp.float32)
    hx2_ref[...] = hx2
    g2_ref[...] = hx2 * dinv_ref[...]


def _t2(a0, a1, hx, dinv, invdeg, b1r, W2):
    return pl.pallas_call(
        _t2_body,
        grid=(N // _BLK,),
        in_specs=[
            pl.BlockSpec((_BLK, HID), lambda i: (i, 0)),
            pl.BlockSpec((_BLK, HID), lambda i: (i, 0)),
            pl.BlockSpec((_BLK, HID), lambda i: (i, 0)),
            pl.BlockSpec((_BLK, 1), lambda i: (i, 0)),
            pl.BlockSpec((_BLK, 1), lambda i: (i, 0)),
            pl.BlockSpec((1, HID), lambda i: (0, 0)),
            pl.BlockSpec((HID, HID), lambda i: (0, 0)),
        ],
        out_specs=[
            pl.BlockSpec((_BLK, HID), lambda i: (i, 0)),
            pl.BlockSpec((_BLK, HID), lambda i: (i, 0)),
        ],
        out_shape=[
            jax.ShapeDtypeStruct((N, HID), jnp.float32),
            jax.ShapeDtypeStruct((N, HID), jnp.float32),
        ],
    )(a0, a1, hx, dinv, invdeg, b1r, W2)


def _t3_body(a0_ref, a1_ref, hx_ref, dinv_ref, invdeg_ref, b_ref,
             A0_ref, A1_ref, wf1_ref, bf1_ref, wf2_ref, bf2_ref,
             out_ref, pool_ref):
    k = pl.program_id(0)
    h2 = dinv_ref[...] * (a0_ref[...] + a1_ref[...]) \
        + hx_ref[...] * invdeg_ref[...] + b_ref[...]
    h2 = jnp.maximum(h2, 0.0)
    A = A0_ref[...] + A1_ref[...]
    part = lax.dot_general(A, h2, (((0,), (0,)), ((), ())),
                           preferred_element_type=jnp.float32)

    @pl.when(k == 0)
    def _():
        pool_ref[...] = part

    @pl.when(k > 0)
    def _():
        pool_ref[...] = pool_ref[...] + part

    @pl.when(k == N // _BLK - 1)
    def _():
        p = pool_ref[...]
        acc = bf1_ref[...]
        for band in range(BANDS):
            acc = acc + jnp.dot(p[band * B:(band + 1) * B],
                                wf1_ref[pl.ds(band * HID, HID), :],
                                preferred_element_type=jnp.float32)
        h1h = jnp.maximum(acc, 0.0)
        out_ref[...] = jnp.dot(h1h, wf2_ref[...],
                               preferred_element_type=jnp.float32) \
            + bf2_ref[...]


def _t3(a0, a1, hx2, dinv, invdeg, b2r, A0, A1, Wf1, bf1r, Wf2p, bf2r):
    return pl.pallas_call(
        _t3_body,
        grid=(N // _BLK,),
        in_specs=[
            pl.BlockSpec((_BLK, HID), lambda k: (k, 0)),
            pl.BlockSpec((_BLK, HID), lambda k: (k, 0)),
            pl.BlockSpec((_BLK, HID), lambda k: (k, 0)),
            pl.BlockSpec((_BLK, 1), lambda k: (k, 0)),
            pl.BlockSpec((_BLK, 1), lambda k: (k, 0)),
            pl.BlockSpec((1, HID), lambda k: (0, 0)),
            pl.BlockSpec((_BLK, AROWS), lambda k: (k, 0)),
            pl.BlockSpec((_BLK, AROWS), lambda k: (k, 0)),
            pl.BlockSpec((BANDS * HID, HID), lambda k: (0, 0)),
            pl.BlockSpec((B, HID), lambda k: (0, 0)),
            pl.BlockSpec((HID, HID), lambda k: (0, 0)),
            pl.BlockSpec((B, HID), lambda k: (0, 0)),
        ],
        out_specs=pl.BlockSpec((B, HID), lambda k: (0, 0)),
        out_shape=jax.ShapeDtypeStruct((B, HID), jnp.float32),
        scratch_shapes=[pltpu.VMEM((AROWS, HID), jnp.float32)],
    )(a0, a1, hx2, dinv, invdeg, b2r, A0, A1, Wf1, bf1r, Wf2p, bf2r)


def kernel(x, edge_index, batch, d_row, d_col, d_val, pool_seg,
           W1, b1, W2, b2, Wf1, bf1, Wf2, bf2):
    row, col = edge_index[0], edge_index[1]

    npad_e = E_PAD - E
    pad_i = jnp.arange(npad_e, dtype=jnp.int32)
    row_pad = jnp.concatenate([row, N + (pad_i % NTRASH)])
    col_pad = jnp.concatenate([col, (pad_i * 79) % N])
    row_p = row_pad.reshape(ECH, 128)
    ec = jnp.stack([row_pad.reshape(NCHUNK, EC),
                    col_pad.reshape(NCHUNK, EC)], axis=1)

    npad_m = M_PAD - M
    zpad = jnp.zeros((npad_m,), jnp.int32)
    drow_p = jnp.concatenate([d_row, zpad]).reshape(MCH, 128)
    dcol_p = jnp.concatenate([d_col, zpad]).reshape(MCH, 128)
    dval_p = jnp.concatenate([d_val, zpad.astype(jnp.float32)]).reshape(MCH, 128)

    ps2 = (pool_seg % BANDS) * B + pool_seg // BANDS

    deg = _deg_kernel(row_p)

    Ap = _abuild_kernel(drow_p, dcol_p, dval_p, ps2)
    A0 = Ap[:AROWS * N].reshape(N, AROWS)
    A1 = Ap[AROWS * N:].reshape(N, AROWS)

    hx1, g1, dinv2, invdeg2 = _t1(x, W1, deg.reshape(N, 1))
    agg1 = _agg_kernel(ec, g1)
    hx2, g2 = _t2(agg1[0], agg1[1], hx1, dinv2, invdeg2,
                  b1.reshape(1, HID), W2)
    agg2 = _agg_kernel(ec, g2)

    Wf2p = jnp.pad(Wf2, ((0, 0), (0, HID - Wf2.shape[1])))
    bf2r = jnp.pad(bf2, (0, HID - bf2.shape[0])).reshape(1, HID)
    bf2b = jnp.broadcast_to(bf2r, (B, HID))
    bf1r = jnp.broadcast_to(bf1.reshape(1, HID), (B, HID))

    outp = _t3(agg2[0], agg2[1], hx2, dinv2, invdeg2, b2.reshape(1, HID),
               A0, A1, Wf1, bf1r, Wf2p, bf2b)
    return outp[:, :Wf2.shape[1]]


# speedup vs baseline: 25.3420x; 1.0162x over previous
import functools

import jax
import jax.numpy as jnp
from jax import lax
from jax.experimental import pallas as pl
from jax.experimental.pallas import tpu as pltpu
from jax.experimental.pallas import tpu_sc as plsc

N = 10000
E = 320000
HID = 128
B = 8
BANDS = 3
M = 960000
R = BANDS * N
NSEG = BANDS * B

NTILES = 16
NCORES = 2

ECH = 2560
E_PAD = ECH * 128
ECH_T = ECH // (NCORES * NTILES)
MCH = 7680
M_PAD = MCH * 128
MCH_T = MCH // (NCORES * NTILES)
MBLK = 48

NTRASH = 16
AROWS = 32

_mesh = plsc.VectorSubcoreMesh(core_axis_name="c", subcore_axis_name="s")


def _zero_vmem_2d(ref, rows, width):
    z = jnp.zeros((16,), jnp.float32)

    def body(i, _):
        for v in range(width // 16):
            ref[i, pl.ds(v * 16, 16)] = z
        return 0

    lax.fori_loop(0, rows, body, 0)


def _zero_vmem_1d(ref, n):
    z = jnp.zeros((16,), jnp.float32)

    def body(i, _):
        ref[pl.ds(i * 16, 16)] = z
        return 0

    lax.fori_loop(0, n // 16, body, 0)


@functools.partial(
    pl.kernel,
    out_type=jax.ShapeDtypeStruct((N,), jnp.float32),
    mesh=_mesh,
    scratch_types=[
        pltpu.VMEM_SHARED((N + NTRASH,), jnp.float32),
        pltpu.VMEM((ECH // NTILES, 128), jnp.int32),
        pltpu.VMEM((640,), jnp.float32),
        pltpu.VMEM((128,), jnp.float32),
    ],
)
def _deg_kernel(row_hbm, deg_hbm, deg_sh, row_v, buf_a, ones_v):
    c = lax.axis_index("c")
    s = lax.axis_index("s")

    @pl.when(c == 0)
    def _():
        _zero_vmem_1d(buf_a, 640)

        @pl.when(s < 15)
        def _():
            pltpu.sync_copy(buf_a, deg_sh.at[pl.ds(s * 640, 640)])

        @pl.when(s == 15)
        def _():
            pltpu.sync_copy(buf_a.at[pl.ds(0, 416)],
                            deg_sh.at[pl.ds(9600, 416)])

        one = jnp.full((16,), 1.0, jnp.float32)
        for v in range(8):
            ones_v[pl.ds(v * 16, 16)] = one
        plsc.subcore_barrier()

        nch = ECH // NTILES
        pltpu.sync_copy(row_hbm.at[pl.ds(s * nch, nch)], row_v)

        def scat(j, _):
            pltpu.sync_copy(ones_v, deg_sh.at[row_v.at[j]], add=True)
            return 0

        lax.fori_loop(0, nch, scat, 0)
        plsc.subcore_barrier()

        @pl.when(s < 15)
        def _():
            pltpu.sync_copy(deg_sh.at[pl.ds(s * 640, 640)], buf_a)
            pltpu.sync_copy(buf_a, deg_hbm.at[pl.ds(s * 640, 640)])

        @pl.when(s == 15)
        def _():
            pltpu.sync_copy(deg_sh.at[pl.ds(9600, 400)],
                            buf_a.at[pl.ds(0, 400)])
            pltpu.sync_copy(buf_a.at[pl.ds(0, 400)],
                            deg_hbm.at[pl.ds(9600, 400)])


_NBLK = MCH_T // MBLK


@functools.partial(
    pl.kernel,
    out_type=jax.ShapeDtypeStruct((NCORES * AROWS * N,), jnp.float32),
    mesh=_mesh,
    scratch_types=[
        pltpu.VMEM_SHARED((AROWS * N,), jnp.float32),
        pltpu.VMEM_SHARED((R,), jnp.int32),
        pltpu.VMEM((MBLK, 128), jnp.int32),
        pltpu.VMEM((MBLK, 128), jnp.int32),
        pltpu.VMEM((MBLK, 128), jnp.int32),
        pltpu.VMEM((MBLK, 128), jnp.int32),
        pltpu.VMEM((MBLK, 128), jnp.float32),
        pltpu.VMEM((MBLK, 128), jnp.float32),
        pltpu.VMEM((MBLK, 128), jnp.int32),
        pltpu.VMEM((1880,), jnp.int32),
        pltpu.VMEM((2048,), jnp.float32),
        pltpu.SemaphoreType.DMA,
        pltpu.SemaphoreType.DMA,
        pltpu.SemaphoreType.DMA,
    ],
)
def _abuild_kernel(drow_hbm, dcol_hbm, dval_hbm, ps2_hbm, out_hbm,
                   a_sh, ps2_sh, dr0, dr1, dc0, dc1, dv0, dv1, seg_v, tbuf,
                   zbuf, semL0, semL1, semS):
    c = lax.axis_index("c")
    s = lax.axis_index("s")
    w = c * NTILES + s
    drb = (dr0, dr1)
    dcb = (dc0, dc1)
    dvb = (dv0, dv1)
    semL = (semL0, semL1)

    @pl.when(s < 15)
    def _():
        pltpu.sync_copy(ps2_hbm.at[pl.ds(s * 1880, 1880)], tbuf)
        pltpu.sync_copy(tbuf, ps2_sh.at[pl.ds(s * 1880, 1880)])

    @pl.when(s == 15)
    def _():
        pltpu.sync_copy(ps2_hbm.at[pl.ds(28200, 1800)],
                        tbuf.at[pl.ds(0, 1800)])
        pltpu.sync_copy(tbuf.at[pl.ds(0, 1800)],
                        ps2_sh.at[pl.ds(28200, 1800)])

    _zero_vmem_1d(zbuf, 2048)
    nwords = AROWS * N // NTILES

    def zb(i, _):
        pltpu.sync_copy(zbuf.at[pl.ds(0, 2000)],
                        a_sh.at[pl.ds(s * nwords + i * 2000, 2000)])
        return 0

    lax.fori_loop(0, nwords // 2000, zb, 0)
    plsc.subcore_barrier()

    def issue_loads(b, blk):
        cbase = w * MCH_T + blk * MBLK
        pltpu.async_copy(drow_hbm.at[pl.ds(cbase, MBLK)], drb[b], semL[b])
        pltpu.async_copy(dcol_hbm.at[pl.ds(cbase, MBLK)], dcb[b], semL[b])
        pltpu.async_copy(dval_hbm.at[pl.ds(cbase, MBLK)], dvb[b], semL[b])

    def wait_loads(b):
        pltpu.make_async_copy(drow_hbm.at[pl.ds(0, MBLK)], drb[b],
                              semL[b]).wait()
        pltpu.make_async_copy(dcol_hbm.at[pl.ds(0, MBLK)], dcb[b],
                              semL[b]).wait()
        pltpu.make_async_copy(dval_hbm.at[pl.ds(0, MBLK)], dvb[b],
                              semL[b]).wait()

    issue_loads(0, 0)
    for blk in range(_NBLK):
        b = blk % 2
        wait_loads(b)
        if blk + 1 < _NBLK:
            issue_loads(1 - b, blk + 1)

        def chunk(j, _):
            pltpu.sync_copy(ps2_sh.at[drb[b].at[j]], seg_v.at[j])
            for v in range(8):
                seg = seg_v[j, pl.ds(v * 16, 16)]
                dc = dcb[b][j, pl.ds(v * 16, 16)]
                drb[b][j, pl.ds(v * 16, 16)] = dc * AROWS + seg
            pltpu.async_copy(dvb[b].at[j], a_sh.at[drb[b].at[j]], semS,
                             add=True)
            return 0

        lax.fori_loop(0, MBLK, chunk, 0)

        def drain(j, _):
            pltpu.make_async_copy(dvb[b].at[0], a_sh.at[drb[b].at[0]],
                                  semS).wait()
            return 0

        lax.fori_loop(0, MBLK, drain, 0)

    plsc.subcore_barrier()

    def outcp(i, _):
        off = s * nwords + i * 2000
        pltpu.sync_copy(a_sh.at[pl.ds(off, 2000)], zbuf.at[pl.ds(0, 2000)])
        pltpu.sync_copy(zbuf.at[pl.ds(0, 2000)],
                        out_hbm.at[pl.ds(c * (AROWS * N) + off, 2000)])
        return 0

    lax.fori_loop(0, nwords // 2000, outcp, 0)


_AGG_ROWS_T = 640
EC = 64
NCHUNK = E_PAD // EC
NCHUNK_T = NCHUNK // (NCORES * NTILES)


@functools.partial(
    pl.kernel,
    out_type=jax.ShapeDtypeStruct((NCORES, N, HID), jnp.float32),
    mesh=_mesh,
    scratch_types=[
        pltpu.VMEM_SHARED((N + NTRASH, HID), jnp.float32),
        pltpu.VMEM((2, EC), jnp.int32),
        pltpu.VMEM((2, EC), jnp.int32),
        pltpu.VMEM((2, EC), jnp.int32),
        pltpu.VMEM((EC, HID), jnp.float32),
        pltpu.VMEM((EC, HID), jnp.float32),
        pltpu.VMEM((80, HID), jnp.float32),
        pltpu.SemaphoreType.DMA,
        pltpu.SemaphoreType.DMA,
        pltpu.SemaphoreType.DMA,
        pltpu.SemaphoreType.DMA,
        pltpu.SemaphoreType.DMA,
    ],
)
def _agg_kernel(ec_hbm, g_hbm, out_hbm, agg_sh, ec0, ec1, sidx, gb0, gb1,
                zbuf, semi0, semi1, semg0, semg1, semS):
    c = lax.axis_index("c")
    s = lax.axis_index("s")
    w = c * NTILES + s
    ecb = (ec0, ec1)
    gbufs = (gb0, gb1)
    semi = (semi0, semi1)
    semg = (semg0, semg1)
    cbase = w * NCHUNK_T

    _zero_vmem_2d(zbuf, 80, HID)
    rbase = s * _AGG_ROWS_T

    @pl.when(s < 15)
    def _():
        for i in range(8):
            pltpu.sync_copy(zbuf, agg_sh.at[pl.ds(rbase + i * 80, 80)])

    @pl.when(s == 15)
    def _():
        for i in range(5):
            pltpu.sync_copy(zbuf, agg_sh.at[pl.ds(rbase + i * 80, 80)])
        pltpu.sync_copy(zbuf.at[pl.ds(0, 16)],
                        agg_sh.at[pl.ds(rbase + 400, 16)])

    plsc.subcore_barrier()

    def issue_idx(b, j):
        pltpu.async_copy(ec_hbm.at[cbase + j], ecb[b], semi[b])

    def wait_idx(b):
        pltpu.make_async_copy(ec_hbm.at[0], ecb[b], semi[b]).wait()

    def issue_gather(b, j):
        del j
        pltpu.async_copy(g_hbm.at[ecb[b].at[1]], gbufs[b], semg[b])

    def wait_gather(b):
        pltpu.make_async_copy(g_hbm.at[pl.ds(0, EC)], gbufs[b],
                              semg[b]).wait()

    def issue_scat(b):
        pltpu.async_copy(gbufs[b], agg_sh.at[sidx.at[b]], semS, add=True)

    def drain_scat(b):
        pltpu.make_async_copy(gbufs[b], agg_sh.at[sidx.at[b]], semS).wait()

    issue_idx(0, 0)
    issue_idx(1, 1)
    wait_idx(0)
    issue_gather(0, 0)

    def outer(i, _):
        for b in range(2):
            j = i * 2 + b
            nb = 1 - b
            wait_gather(b)
            for v in range(EC // 16):
                sidx[b, pl.ds(v * 16, 16)] = ecb[b][0, pl.ds(v * 16, 16)]

            @pl.when(j + 2 < NCHUNK_T)
            def _():
                issue_idx(b, j + 2)

            @pl.when(j >= 1)
            def _():
                drain_scat(nb)

            issue_scat(b)

            @pl.when(j + 1 < NCHUNK_T)
            def _():
                wait_idx(nb)
                issue_gather(nb, j + 1)
        return 0

    lax.fori_loop(0, NCHUNK_T // 2, outer, 0)
    drain_scat(1)
    plsc.subcore_barrier()

    def outcp(nrows):
        for i in range(nrows // 80):
            pltpu.sync_copy(agg_sh.at[pl.ds(rbase + i * 80, 80)], zbuf)
            pltpu.sync_copy(zbuf, out_hbm.at[c, pl.ds(rbase + i * 80, 80)])

    @pl.when(s < 15)
    def _():
        outcp(_AGG_ROWS_T)

    @pl.when(s == 15)
    def _():
        outcp(N - 15 * _AGG_ROWS_T)


_BLK = 1000


def _t1_body(x_ref, w_ref, deg_ref, hx_ref, g_ref, dinv_ref, invdeg_ref):
    hx = jnp.dot(x_ref[...], w_ref[...], preferred_element_type=jnp.float32)
    hx_ref[...] = hx
    deg = deg_ref[...] + 1.0
    dinv = lax.rsqrt(deg)
    dinv_ref[...] = dinv
    invdeg_ref[...] = 1.0 / deg
    g_ref[...] = hx * dinv


def _t1(x, W1, deg2):
    return pl.pallas_call(
        _t1_body,
        grid=(N // _BLK,),
        in_specs=[
            pl.BlockSpec((_BLK, HID), lambda i: (i, 0)),
            pl.BlockSpec((HID, HID), lambda i: (0, 0)),
            pl.BlockSpec((_BLK, 1), lambda i: (i, 0)),
        ],
        out_specs=[
            pl.BlockSpec((_BLK, HID), lambda i: (i, 0)),
            pl.BlockSpec((_BLK, HID), lambda i: (i, 0)),
            pl.BlockSpec((_BLK, 1), lambda i: (i, 0)),
            pl.BlockSpec((_BLK, 1), lambda i: (i, 0)),
        ],
        out_shape=[
            jax.ShapeDtypeStruct((N, HID), jnp.float32),
            jax.ShapeDtypeStruct((N, HID), jnp.float32),
            jax.ShapeDtypeStruct((N, 1), jnp.float32),
            jax.ShapeDtypeStruct((N, 1), jnp.float32),
        ],
    )(x, W1, deg2)


def _t2_body(a_ref, hx_ref, dinv_ref, invdeg_ref, b_ref, w_ref,
             hx2_ref, g2_ref):
    h1 = dinv_ref[...] * (a_ref[0] + a_ref[1]) \
        + hx_ref[...] * invdeg_ref[...] + b_ref[...]
    h1 = jnp.maximum(h1, 0.0)
    hx2 = jnp.dot(h1, w_ref[...], preferred_element_type=jnp.float32)
    hx2_ref[...] = hx2
    g2_ref[...] = hx2 * dinv_ref[...]


def _t2(agg, hx, dinv, invdeg, b1r, W2):
    return pl.pallas_call(
        _t2_body,
        grid=(N // _BLK,),
        in_specs=[
            pl.BlockSpec((NCORES, _BLK, HID), lambda i: (0, i, 0)),
            pl.BlockSpec((_BLK, HID), lambda i: (i, 0)),
            pl.BlockSpec((_BLK, 1), lambda i: (i, 0)),
            pl.BlockSpec((_BLK, 1), lambda i: (i, 0)),
            pl.BlockSpec((1, HID), lambda i: (0, 0)),
            pl.BlockSpec((HID, HID), lambda i: (0, 0)),
        ],
        out_specs=[
            pl.BlockSpec((_BLK, HID), lambda i: (i, 0)),
            pl.BlockSpec((_BLK, HID), lambda i: (i, 0)),
        ],
        out_shape=[
            jax.ShapeDtypeStruct((N, HID), jnp.float32),
            jax.ShapeDtypeStruct((N, HID), jnp.float32),
        ],
    )(agg, hx, dinv, invdeg, b1r, W2)


def _t3_body(a_ref, hx_ref, dinv_ref, invdeg_ref, b_ref,
             A_ref, wf1_ref, bf1_ref, wf2_ref, bf2_ref,
             out_ref, pool_ref):
    k = pl.program_id(0)
    h2 = dinv_ref[...] * (a_ref[0] + a_ref[1]) \
        + hx_ref[...] * invdeg_ref[...] + b_ref[...]
    h2 = jnp.maximum(h2, 0.0)
    A = A_ref[0] + A_ref[1]
    part = lax.dot_general(A, h2, (((0,), (0,)), ((), ())),
                           preferred_element_type=jnp.float32)

    @pl.when(k == 0)
    def _():
        pool_ref[...] = part

    @pl.when(k > 0)
    def _():
        pool_ref[...] = pool_ref[...] + part

    @pl.when(k == N // _BLK - 1)
    def _():
        p = pool_ref[...]
        acc = bf1_ref[...]
        for band in range(BANDS):
            acc = acc + jnp.dot(p[band * B:(band + 1) * B],
                                wf1_ref[pl.ds(band * HID, HID), :],
                                preferred_element_type=jnp.float32)
        h1h = jnp.maximum(acc, 0.0)
        out_ref[...] = jnp.dot(h1h, wf2_ref[...],
                               preferred_element_type=jnp.float32) \
            + bf2_ref[...]


def _t3(agg, hx2, dinv, invdeg, b2r, Ap3, Wf1, bf1r, Wf2p, bf2r):
    return pl.pallas_call(
        _t3_body,
        grid=(N // _BLK,),
        in_specs=[
            pl.BlockSpec((NCORES, _BLK, HID), lambda k: (0, k, 0)),
            pl.BlockSpec((_BLK, HID), lambda k: (k, 0)),
            pl.BlockSpec((_BLK, 1), lambda k: (k, 0)),
            pl.BlockSpec((_BLK, 1), lambda k: (k, 0)),
            pl.BlockSpec((1, HID), lambda k: (0, 0)),
            pl.BlockSpec((NCORES, _BLK, AROWS), lambda k: (0, k, 0)),
            pl.BlockSpec((BANDS * HID, HID), lambda k: (0, 0)),
            pl.BlockSpec((B, HID), lambda k: (0, 0)),
            pl.BlockSpec((HID, HID), lambda k: (0, 0)),
            pl.BlockSpec((B, HID), lambda k: (0, 0)),
        ],
        out_specs=pl.BlockSpec((B, HID), lambda k: (0, 0)),
        out_shape=jax.ShapeDtypeStruct((B, HID), jnp.float32),
        scratch_shapes=[pltpu.VMEM((AROWS, HID), jnp.float32)],
    )(agg, hx2, dinv, invdeg, b2r, Ap3, Wf1, bf1r, Wf2p, bf2r)


def kernel(x, edge_index, batch, d_row, d_col, d_val, pool_seg,
           W1, b1, W2, b2, Wf1, bf1, Wf2, bf2):
    row, col = edge_index[0], edge_index[1]

    npad_e = E_PAD - E
    pad_i = jnp.arange(npad_e, dtype=jnp.int32)
    row_pad = jnp.concatenate([row, N + (pad_i % NTRASH)])
    col_pad = jnp.concatenate([col, (pad_i * 79) % N])
    row_p = row_pad.reshape(ECH, 128)
    ec = jnp.stack([row_pad.reshape(NCHUNK, EC),
                    col_pad.reshape(NCHUNK, EC)], axis=1)

    npad_m = M_PAD - M
    zpad = jnp.zeros((npad_m,), jnp.int32)
    drow_p = jnp.concatenate([d_row, zpad]).reshape(MCH, 128)
    dcol_p = jnp.concatenate([d_col, zpad]).reshape(MCH, 128)
    dval_p = jnp.concatenate([d_val, zpad.astype(jnp.float32)]).reshape(MCH, 128)

    ps2 = (pool_seg % BANDS) * B + pool_seg // BANDS

    deg = _deg_kernel(row_p)

    Ap = _abuild_kernel(drow_p, dcol_p, dval_p, ps2)
    Ap3 = Ap.reshape(NCORES, N, AROWS)

    hx1, g1, dinv2, invdeg2 = _t1(x, W1, deg.reshape(N, 1))
    agg1 = _agg_kernel(ec, g1)
    hx2, g2 = _t2(agg1, hx1, dinv2, invdeg2, b1.reshape(1, HID), W2)
    agg2 = _agg_kernel(ec, g2)

    Wf2p = jnp.pad(Wf2, ((0, 0), (0, HID - Wf2.shape[1])))
    bf2r = jnp.pad(bf2, (0, HID - bf2.shape[0])).reshape(1, HID)
    bf2b = jnp.broadcast_to(bf2r, (B, HID))
    bf1r = jnp.broadcast_to(bf1.reshape(1, HID), (B, HID))

    outp = _t3(agg2, hx2, dinv2, invdeg2, b2.reshape(1, HID),
               Ap3, Wf1, bf1r, Wf2p, bf2b)
    return outp[:, :Wf2.shape[1]]


# speedup vs baseline: 25.3424x; 1.0000x over previous
import functools

import jax
import jax.numpy as jnp
from jax import lax
from jax.experimental import pallas as pl
from jax.experimental.pallas import tpu as pltpu
from jax.experimental.pallas import tpu_sc as plsc

N = 10000
E = 320000
HID = 128
B = 8
BANDS = 3
M = 960000
R = BANDS * N
NSEG = BANDS * B

NTILES = 16
NCORES = 2

ECH = 2560
E_PAD = ECH * 128
ECH_T = ECH // (NCORES * NTILES)
MCH = 7680
M_PAD = MCH * 128
MCH_T = MCH // (NCORES * NTILES)
MBLK = 48

NTRASH = 16
AROWS = 32

_mesh = plsc.VectorSubcoreMesh(core_axis_name="c", subcore_axis_name="s")


def _zero_vmem_2d(ref, rows, width):
    z = jnp.zeros((16,), jnp.float32)

    def body(i, _):
        for v in range(width // 16):
            ref[i, pl.ds(v * 16, 16)] = z
        return 0

    lax.fori_loop(0, rows, body, 0)


def _zero_vmem_1d(ref, n):
    z = jnp.zeros((16,), jnp.float32)

    def body(i, _):
        ref[pl.ds(i * 16, 16)] = z
        return 0

    lax.fori_loop(0, n // 16, body, 0)


@functools.partial(
    pl.kernel,
    out_type=jax.ShapeDtypeStruct((N,), jnp.float32),
    mesh=_mesh,
    scratch_types=[
        pltpu.VMEM_SHARED((N + NTRASH,), jnp.float32),
        pltpu.VMEM((ECH // NTILES, 128), jnp.int32),
        pltpu.VMEM((640,), jnp.float32),
        pltpu.VMEM((128,), jnp.float32),
    ],
)
def _deg_kernel(row_hbm, deg_hbm, deg_sh, row_v, buf_a, ones_v):
    c = lax.axis_index("c")
    s = lax.axis_index("s")

    @pl.when(c == 0)
    def _():
        _zero_vmem_1d(buf_a, 640)

        @pl.when(s < 15)
        def _():
            pltpu.sync_copy(buf_a, deg_sh.at[pl.ds(s * 640, 640)])

        @pl.when(s == 15)
        def _():
            pltpu.sync_copy(buf_a.at[pl.ds(0, 416)],
                            deg_sh.at[pl.ds(9600, 416)])

        one = jnp.full((16,), 1.0, jnp.float32)
        for v in range(8):
            ones_v[pl.ds(v * 16, 16)] = one
        plsc.subcore_barrier()

        nch = ECH // NTILES
        pltpu.sync_copy(row_hbm.at[pl.ds(s * nch, nch)], row_v)

        def scat(j, _):
            pltpu.sync_copy(ones_v, deg_sh.at[row_v.at[j]], add=True)
            return 0

        lax.fori_loop(0, nch, scat, 0)
        plsc.subcore_barrier()

        @pl.when(s < 15)
        def _():
            pltpu.sync_copy(deg_sh.at[pl.ds(s * 640, 640)], buf_a)
            pltpu.sync_copy(buf_a, deg_hbm.at[pl.ds(s * 640, 640)])

        @pl.when(s == 15)
        def _():
            pltpu.sync_copy(deg_sh.at[pl.ds(9600, 400)],
                            buf_a.at[pl.ds(0, 400)])
            pltpu.sync_copy(buf_a.at[pl.ds(0, 400)],
                            deg_hbm.at[pl.ds(9600, 400)])


_NBLK = MCH_T // MBLK


@functools.partial(
    pl.kernel,
    out_type=jax.ShapeDtypeStruct((NCORES * AROWS * N,), jnp.float32),
    mesh=_mesh,
    scratch_types=[
        pltpu.VMEM_SHARED((AROWS * N,), jnp.float32),
        pltpu.VMEM_SHARED((R,), jnp.int32),
        pltpu.VMEM((MBLK, 128), jnp.int32),
        pltpu.VMEM((MBLK, 128), jnp.int32),
        pltpu.VMEM((MBLK, 128), jnp.int32),
        pltpu.VMEM((MBLK, 128), jnp.int32),
        pltpu.VMEM((MBLK, 128), jnp.float32),
        pltpu.VMEM((MBLK, 128), jnp.float32),
        pltpu.VMEM((MBLK, 128), jnp.int32),
        pltpu.VMEM((1880,), jnp.int32),
        pltpu.VMEM((2048,), jnp.float32),
        pltpu.SemaphoreType.DMA,
        pltpu.SemaphoreType.DMA,
        pltpu.SemaphoreType.DMA,
    ],
)
def _abuild_kernel(drow_hbm, dcol_hbm, dval_hbm, ps2_hbm, out_hbm,
                   a_sh, ps2_sh, dr0, dr1, dc0, dc1, dv0, dv1, seg_v, tbuf,
                   zbuf, semL0, semL1, semS):
    c = lax.axis_index("c")
    s = lax.axis_index("s")
    w = c * NTILES + s
    drb = (dr0, dr1)
    dcb = (dc0, dc1)
    dvb = (dv0, dv1)
    semL = (semL0, semL1)

    @pl.when(s < 15)
    def _():
        pltpu.sync_copy(ps2_hbm.at[pl.ds(s * 1880, 1880)], tbuf)
        pltpu.sync_copy(tbuf, ps2_sh.at[pl.ds(s * 1880, 1880)])

    @pl.when(s == 15)
    def _():
        pltpu.sync_copy(ps2_hbm.at[pl.ds(28200, 1800)],
                        tbuf.at[pl.ds(0, 1800)])
        pltpu.sync_copy(tbuf.at[pl.ds(0, 1800)],
                        ps2_sh.at[pl.ds(28200, 1800)])

    _zero_vmem_1d(zbuf, 2048)
    nwords = AROWS * N // NTILES

    def zb(i, _):
        pltpu.sync_copy(zbuf.at[pl.ds(0, 2000)],
                        a_sh.at[pl.ds(s * nwords + i * 2000, 2000)])
        return 0

    lax.fori_loop(0, nwords // 2000, zb, 0)
    plsc.subcore_barrier()

    def issue_loads(b, blk):
        cbase = w * MCH_T + blk * MBLK
        pltpu.async_copy(drow_hbm.at[pl.ds(cbase, MBLK)], drb[b], semL[b])
        pltpu.async_copy(dcol_hbm.at[pl.ds(cbase, MBLK)], dcb[b], semL[b])
        pltpu.async_copy(dval_hbm.at[pl.ds(cbase, MBLK)], dvb[b], semL[b])

    def wait_loads(b):
        pltpu.make_async_copy(drow_hbm.at[pl.ds(0, MBLK)], drb[b],
                              semL[b]).wait()
        pltpu.make_async_copy(dcol_hbm.at[pl.ds(0, MBLK)], dcb[b],
                              semL[b]).wait()
        pltpu.make_async_copy(dval_hbm.at[pl.ds(0, MBLK)], dvb[b],
                              semL[b]).wait()

    issue_loads(0, 0)
    for blk in range(_NBLK):
        b = blk % 2
        wait_loads(b)
        if blk + 1 < _NBLK:
            issue_loads(1 - b, blk + 1)

        def chunk(j, _):
            pltpu.sync_copy(ps2_sh.at[drb[b].at[j]], seg_v.at[j])
            for v in range(8):
                seg = seg_v[j, pl.ds(v * 16, 16)]
                dc = dcb[b][j, pl.ds(v * 16, 16)]
                drb[b][j, pl.ds(v * 16, 16)] = dc * AROWS + seg
            pltpu.async_copy(dvb[b].at[j], a_sh.at[drb[b].at[j]], semS,
                             add=True)
            return 0

        lax.fori_loop(0, MBLK, chunk, 0)

        def drain(j, _):
            pltpu.make_async_copy(dvb[b].at[0], a_sh.at[drb[b].at[0]],
                                  semS).wait()
            return 0

        lax.fori_loop(0, MBLK, drain, 0)

    plsc.subcore_barrier()

    def outcp(i, _):
        off = s * nwords + i * 2000
        pltpu.sync_copy(a_sh.at[pl.ds(off, 2000)], zbuf.at[pl.ds(0, 2000)])
        pltpu.sync_copy(zbuf.at[pl.ds(0, 2000)],
                        out_hbm.at[pl.ds(c * (AROWS * N) + off, 2000)])
        return 0

    lax.fori_loop(0, nwords // 2000, outcp, 0)


_AGG_ROWS_T = 640
EC = 64
NCHUNK = E_PAD // EC
NCHUNK_T = NCHUNK // (NCORES * NTILES)


@functools.partial(
    pl.kernel,
    out_type=jax.ShapeDtypeStruct((NCORES, N, HID), jnp.float32),
    mesh=_mesh,
    scratch_types=[
        pltpu.VMEM_SHARED((N + NTRASH, HID), jnp.float32),
        pltpu.VMEM((2, EC), jnp.int32),
        pltpu.VMEM((2, EC), jnp.int32),
        pltpu.VMEM((2, EC), jnp.int32),
        pltpu.VMEM((EC, HID), jnp.float32),
        pltpu.VMEM((EC, HID), jnp.float32),
        pltpu.VMEM((80, HID), jnp.float32),
        pltpu.SemaphoreType.DMA,
        pltpu.SemaphoreType.DMA,
        pltpu.SemaphoreType.DMA,
        pltpu.SemaphoreType.DMA,
        pltpu.SemaphoreType.DMA,
    ],
)
def _agg_kernel(row_hbm, col_hbm, g_hbm, out_hbm, agg_sh, rv, cv, sidx,
                gb0, gb1, zbuf, semi0, semi1, semg0, semg1, semS):
    c = lax.axis_index("c")
    s = lax.axis_index("s")
    w = c * NTILES + s
    gbufs = (gb0, gb1)
    semi = (semi0, semi1)
    semg = (semg0, semg1)
    cbase = w * NCHUNK_T

    _zero_vmem_2d(zbuf, 80, HID)
    rbase = s * _AGG_ROWS_T

    @pl.when(s < 15)
    def _():
        for i in range(8):
            pltpu.sync_copy(zbuf, agg_sh.at[pl.ds(rbase + i * 80, 80)])

    @pl.when(s == 15)
    def _():
        for i in range(5):
            pltpu.sync_copy(zbuf, agg_sh.at[pl.ds(rbase + i * 80, 80)])
        pltpu.sync_copy(zbuf.at[pl.ds(0, 16)],
                        agg_sh.at[pl.ds(rbase + 400, 16)])

    plsc.subcore_barrier()

    def issue_idx(b, j):
        pltpu.async_copy(row_hbm.at[cbase + j], rv.at[b], semi[b])
        pltpu.async_copy(col_hbm.at[cbase + j], cv.at[b], semi[b])

    def wait_idx(b):
        pltpu.make_async_copy(row_hbm.at[0], rv.at[b], semi[b]).wait()
        pltpu.make_async_copy(col_hbm.at[0], cv.at[b], semi[b]).wait()

    def issue_gather(b, j):
        del j
        pltpu.async_copy(g_hbm.at[cv.at[b]], gbufs[b], semg[b])

    def wait_gather(b):
        pltpu.make_async_copy(g_hbm.at[pl.ds(0, EC)], gbufs[b],
                              semg[b]).wait()

    def issue_scat(b):
        pltpu.async_copy(gbufs[b], agg_sh.at[sidx.at[b]], semS, add=True)

    def drain_scat(b):
        pltpu.make_async_copy(gbufs[b], agg_sh.at[sidx.at[b]], semS).wait()

    issue_idx(0, 0)
    issue_idx(1, 1)
    wait_idx(0)
    issue_gather(0, 0)

    def outer(i, _):
        for b in range(2):
            j = i * 2 + b
            nb = 1 - b
            wait_gather(b)
            for v in range(EC // 16):
                sidx[b, pl.ds(v * 16, 16)] = rv[b, pl.ds(v * 16, 16)]

            @pl.when(j + 2 < NCHUNK_T)
            def _():
                issue_idx(b, j + 2)

            @pl.when(j >= 1)
            def _():
                drain_scat(nb)

            issue_scat(b)

            @pl.when(j + 1 < NCHUNK_T)
            def _():
                wait_idx(nb)
                issue_gather(nb, j + 1)
        return 0

    lax.fori_loop(0, NCHUNK_T // 2, outer, 0)
    drain_scat(1)
    plsc.subcore_barrier()

    def outcp(nrows):
        for i in range(nrows // 80):
            pltpu.sync_copy(agg_sh.at[pl.ds(rbase + i * 80, 80)], zbuf)
            pltpu.sync_copy(zbuf, out_hbm.at[c, pl.ds(rbase + i * 80, 80)])

    @pl.when(s < 15)
    def _():
        outcp(_AGG_ROWS_T)

    @pl.when(s == 15)
    def _():
        outcp(N - 15 * _AGG_ROWS_T)


_BLK = 1000


def _t1_body(x_ref, w_ref, deg_ref, hx_ref, g_ref, dinv_ref, invdeg_ref):
    hx = jnp.dot(x_ref[...], w_ref[...], preferred_element_type=jnp.float32)
    hx_ref[...] = hx
    deg = deg_ref[...] + 1.0
    dinv = lax.rsqrt(deg)
    dinv_ref[...] = dinv
    invdeg_ref[...] = 1.0 / deg
    g_ref[...] = hx * dinv


def _t1(x, W1, deg2):
    return pl.pallas_call(
        _t1_body,
        grid=(N // _BLK,),
        in_specs=[
            pl.BlockSpec((_BLK, HID), lambda i: (i, 0)),
            pl.BlockSpec((HID, HID), lambda i: (0, 0)),
            pl.BlockSpec((_BLK, 1), lambda i: (i, 0)),
        ],
        out_specs=[
            pl.BlockSpec((_BLK, HID), lambda i: (i, 0)),
            pl.BlockSpec((_BLK, HID), lambda i: (i, 0)),
            pl.BlockSpec((_BLK, 1), lambda i: (i, 0)),
            pl.BlockSpec((_BLK, 1), lambda i: (i, 0)),
        ],
        out_shape=[
            jax.ShapeDtypeStruct((N, HID), jnp.float32),
            jax.ShapeDtypeStruct((N, HID), jnp.float32),
            jax.ShapeDtypeStruct((N, 1), jnp.float32),
            jax.ShapeDtypeStruct((N, 1), jnp.float32),
        ],
    )(x, W1, deg2)


def _t2_body(a_ref, hx_ref, dinv_ref, invdeg_ref, b_ref, w_ref,
             hx2_ref, g2_ref):
    h1 = dinv_ref[...] * (a_ref[0] + a_ref[1]) \
        + hx_ref[...] * invdeg_ref[...] + b_ref[...]
    h1 = jnp.maximum(h1, 0.0)
    hx2 = jnp.dot(h1, w_ref[...], preferred_element_type=jnp.float32)
    hx2_ref[...] = hx2
    g2_ref[...] = hx2 * dinv_ref[...]


def _t2(agg, hx, dinv, invdeg, b1r, W2):
    return pl.pallas_call(
        _t2_body,
        grid=(N // _BLK,),
        in_specs=[
            pl.BlockSpec((NCORES, _BLK, HID), lambda i: (0, i, 0)),
            pl.BlockSpec((_BLK, HID), lambda i: (i, 0)),
            pl.BlockSpec((_BLK, 1), lambda i: (i, 0)),
            pl.BlockSpec((_BLK, 1), lambda i: (i, 0)),
            pl.BlockSpec((1, HID), lambda i: (0, 0)),
            pl.BlockSpec((HID, HID), lambda i: (0, 0)),
        ],
        out_specs=[
            pl.BlockSpec((_BLK, HID), lambda i: (i, 0)),
            pl.BlockSpec((_BLK, HID), lambda i: (i, 0)),
        ],
        out_shape=[
            jax.ShapeDtypeStruct((N, HID), jnp.float32),
            jax.ShapeDtypeStruct((N, HID), jnp.float32),
        ],
    )(agg, hx, dinv, invdeg, b1r, W2)


def _t3_body(a_ref, hx_ref, dinv_ref, invdeg_ref, b_ref,
             A_ref, wf1_ref, bf1_ref, wf2_ref, bf2_ref,
             out_ref, pool_ref):
    k = pl.program_id(0)
    h2 = dinv_ref[...] * (a_ref[0] + a_ref[1]) \
        + hx_ref[...] * invdeg_ref[...] + b_ref[...]
    h2 = jnp.maximum(h2, 0.0)
    A = A_ref[0] + A_ref[1]
    part = lax.dot_general(A, h2, (((0,), (0,)), ((), ())),
                           preferred_element_type=jnp.float32)

    @pl.when(k == 0)
    def _():
        pool_ref[...] = part

    @pl.when(k > 0)
    def _():
        pool_ref[...] = pool_ref[...] + part

    @pl.when(k == N // _BLK - 1)
    def _():
        p = pool_ref[...]
        acc = bf1_ref[...]
        for band in range(BANDS):
            acc = acc + jnp.dot(p[band * B:(band + 1) * B],
                                wf1_ref[pl.ds(band * HID, HID), :],
                                preferred_element_type=jnp.float32)
        h1h = jnp.maximum(acc, 0.0)
        out_ref[...] = jnp.dot(h1h, wf2_ref[...],
                               preferred_element_type=jnp.float32) \
            + bf2_ref[...]


def _t3(agg, hx2, dinv, invdeg, b2r, Ap3, Wf1, bf1r, Wf2p, bf2r):
    return pl.pallas_call(
        _t3_body,
        grid=(N // _BLK,),
        in_specs=[
            pl.BlockSpec((NCORES, _BLK, HID), lambda k: (0, k, 0)),
            pl.BlockSpec((_BLK, HID), lambda k: (k, 0)),
            pl.BlockSpec((_BLK, 1), lambda k: (k, 0)),
            pl.BlockSpec((_BLK, 1), lambda k: (k, 0)),
            pl.BlockSpec((1, HID), lambda k: (0, 0)),
            pl.BlockSpec((NCORES, _BLK, AROWS), lambda k: (0, k, 0)),
            pl.BlockSpec((BANDS * HID, HID), lambda k: (0, 0)),
            pl.BlockSpec((B, HID), lambda k: (0, 0)),
            pl.BlockSpec((HID, HID), lambda k: (0, 0)),
            pl.BlockSpec((B, HID), lambda k: (0, 0)),
        ],
        out_specs=pl.BlockSpec((B, HID), lambda k: (0, 0)),
        out_shape=jax.ShapeDtypeStruct((B, HID), jnp.float32),
        scratch_shapes=[pltpu.VMEM((AROWS, HID), jnp.float32)],
    )(agg, hx2, dinv, invdeg, b2r, Ap3, Wf1, bf1r, Wf2p, bf2r)


def kernel(x, edge_index, batch, d_row, d_col, d_val, pool_seg,
           W1, b1, W2, b2, Wf1, bf1, Wf2, bf2):
    row, col = edge_index[0], edge_index[1]

    npad_e = E_PAD - E
    pad_i = jnp.arange(npad_e, dtype=jnp.int32)
    row_pad = jnp.concatenate([row, N + (pad_i % NTRASH)])
    col_pad = jnp.concatenate([col, (pad_i * 79) % N])
    row_p = row_pad.reshape(ECH, 128)
    rowc = row_pad.reshape(NCHUNK, EC)
    colc = col_pad.reshape(NCHUNK, EC)

    npad_m = M_PAD - M
    zpad = jnp.zeros((npad_m,), jnp.int32)
    drow_p = jnp.concatenate([d_row, zpad]).reshape(MCH, 128)
    dcol_p = jnp.concatenate([d_col, zpad]).reshape(MCH, 128)
    dval_p = jnp.concatenate([d_val, zpad.astype(jnp.float32)]).reshape(MCH, 128)

    ps2 = (pool_seg % BANDS) * B + pool_seg // BANDS

    deg = _deg_kernel(row_p)

    Ap = _abuild_kernel(drow_p, dcol_p, dval_p, ps2)
    Ap3 = Ap.reshape(NCORES, N, AROWS)

    hx1, g1, dinv2, invdeg2 = _t1(x, W1, deg.reshape(N, 1))
    agg1 = _agg_kernel(rowc, colc, g1)
    hx2, g2 = _t2(agg1, hx1, dinv2, invdeg2, b1.reshape(1, HID), W2)
    agg2 = _agg_kernel(rowc, colc, g2)

    Wf2p = jnp.pad(Wf2, ((0, 0), (0, HID - Wf2.shape[1])))
    bf2r = jnp.pad(bf2, (0, HID - bf2.shape[0])).reshape(1, HID)
    bf2b = jnp.broadcast_to(bf2r, (B, HID))
    bf1r = jnp.broadcast_to(bf1.reshape(1, HID), (B, HID))

    outp = _t3(agg2, hx2, dinv2, invdeg2, b2.reshape(1, HID),
               Ap3, Wf1, bf1r, Wf2p, bf2b)
    return outp[:, :Wf2.shape[1]]
